# Initial kernel scaffold; baseline (speedup 1.0000x reference)
import jax
import jax.numpy as jnp
from jax.experimental import pallas as pl


def kernel(voxel_feats, voxel_mask, W_init, b_init, db0_w1, db0_w2, db1_w1, db1_w2, db2_w1, db2_w2, ds0_w, ds1_w, us0_w, us1_w, ub0_w1, ub0_w2, ub1_w1, ub1_w2, out0_w, out1_w, out2_w):
    raise NotImplementedError("write your pallas kernel here")


# speedup vs baseline: 1.2702x; 1.2702x over previous
import jax
import jax.numpy as jnp
from jax.experimental import pallas as pl
from jax.experimental.pallas import tpu as pltpu

_F32 = jnp.float32
_PAD = 128


def _pl_lanes(D):
    S = D + 2
    need = _PAD + S * S + S + 1
    return -(-need // 128) * 128


def _conv3(x, m, w, D, b=None, inorm=False, lrelu=False):
    S = D + 2
    S2 = S * S
    PL = x.shape[2]
    CI = x.shape[1]
    CO = w.shape[1]
    w9 = jnp.transpose(w.reshape(3, 9, CO, CI), (0, 2, 1, 3)).reshape(3, CO, 9 * CI)
    offs = [_PAD + dy * S + dx for dy in (-1, 0, 1) for dx in (-1, 0, 1)]

    def body(x_ref, m_ref, w_ref, *rest):
        if b is not None:
            b_ref, o_ref = rest
        else:
            (o_ref,) = rest
        o_ref[0] = jnp.zeros((CO, PL), _F32)
        o_ref[S - 1] = jnp.zeros((CO, PL), _F32)

        def plane(g, carry):
            s1, s2, nn = carry
            y = jnp.zeros((CO, S2), _F32)
            for dz in range(3):
                v = x_ref[g + dz]
                xc = jnp.concatenate(
                    [jax.lax.slice(v, (0, o), (CI, o + S2)) for o in offs], axis=0)
                y = y + jax.lax.dot(w_ref[dz], xc, preferred_element_type=_F32)
            if b is not None:
                y = y + b_ref[...]
            mv = m_ref[g + 1]
            mm = jax.lax.slice(mv, (0, _PAD), (1, _PAD + S2))
            y = y * mm
            if lrelu and not inorm:
                y = jnp.where(y >= 0, y, 0.2 * y)
            o_ref[g + 1] = jnp.pad(y, ((0, 0), (_PAD, PL - _PAD - S2)))
            if inorm:
                s1 = s1 + jnp.sum(y, axis=1, keepdims=True)
                s2 = s2 + jnp.sum(y * y, axis=1, keepdims=True)
                nn = nn + jnp.sum(mm)
            return (s1, s2, nn)

        init = (jnp.zeros((CO, 1), _F32), jnp.zeros((CO, 1), _F32), _F32(0))
        s1, s2, nn = jax.lax.fori_loop(0, D, plane, init)
        if inorm:
            n = jnp.maximum(nn, 1.0)
            mu = s1 / n
            var = s2 / n - mu * mu
            inv = jax.lax.rsqrt(var + 1e-5)

            def norm(z, _):
                v = (o_ref[z] - mu) * inv * m_ref[z]
                v = jnp.where(v >= 0, v, 0.2 * v) if lrelu else v
                o_ref[z] = v
                return 0

            jax.lax.fori_loop(1, D + 1, norm, 0)

    args = [x, m, w9] + ([b.reshape(CO, 1)] if b is not None else [])
    return pl.pallas_call(
        body, out_shape=jax.ShapeDtypeStruct((S, CO, PL), _F32))(*args)


def _conv3_stream(x, m, w, D, inorm=False, lrelu=False):
    S = D + 2
    S2 = S * S
    PL = x.shape[2]
    CI = x.shape[1]
    CO = w.shape[1]
    w9 = jnp.transpose(w.reshape(3, 9, CO, CI), (0, 2, 1, 3)).reshape(3, CO, 9 * CI)
    offs = [_PAD + dy * S + dx for dy in (-1, 0, 1) for dx in (-1, 0, 1)]

    def body(x0_ref, x1_ref, x2_ref, m_ref, w_ref, o_ref, s1_ref, s2_ref, nn_ref):
        g = pl.program_id(0)

        @pl.when(g == 0)
        def _init():
            s1_ref[...] = jnp.zeros((CO, 1), _F32)
            s2_ref[...] = jnp.zeros((CO, 1), _F32)
            nn_ref[0] = _F32(0)
            o_ref[0] = jnp.zeros((CO, PL), _F32)
            o_ref[S - 1] = jnp.zeros((CO, PL), _F32)

        y = jnp.zeros((CO, S2), _F32)
        for dz, xr in enumerate((x0_ref, x1_ref, x2_ref)):
            v = xr[0]
            xc = jnp.concatenate(
                [jax.lax.slice(v, (0, o), (CI, o + S2)) for o in offs], axis=0)
            y = y + jax.lax.dot(w_ref[dz], xc, preferred_element_type=_F32)
        mv = m_ref[g + 1]
        mm = jax.lax.slice(mv, (0, _PAD), (1, _PAD + S2))
        y = y * mm
        if lrelu and not inorm:
            y = jnp.where(y >= 0, y, 0.2 * y)
        o_ref[g + 1] = jnp.pad(y, ((0, 0), (_PAD, PL - _PAD - S2)))
        if inorm:
            s1_ref[...] += jnp.sum(y, axis=1, keepdims=True)
            s2_ref[...] += jnp.sum(y * y, axis=1, keepdims=True)
            nn_ref[0] += jnp.sum(mm)

            @pl.when(g == D - 1)
            def _finish():
                n = jnp.maximum(nn_ref[0], 1.0)
                mu = s1_ref[...] / n
                var = s2_ref[...] / n - mu * mu
                inv = jax.lax.rsqrt(var + 1e-5)

                def norm(z, _):
                    v = (o_ref[z] - mu) * inv * m_ref[z]
                    if lrelu:
                        v = jnp.where(v >= 0, v, 0.2 * v)
                    o_ref[z] = v
                    return 0

                jax.lax.fori_loop(1, D + 1, norm, 0)

    return pl.pallas_call(
        body,
        grid=(D,),
        in_specs=[
            pl.BlockSpec((1, CI, PL), lambda g: (g, 0, 0)),
            pl.BlockSpec((1, CI, PL), lambda g: (g + 1, 0, 0)),
            pl.BlockSpec((1, CI, PL), lambda g: (g + 2, 0, 0)),
            pl.BlockSpec((S, 1, PL), lambda g: (0, 0, 0)),
            pl.BlockSpec((3, CO, 9 * CI), lambda g: (0, 0, 0)),
        ],
        out_specs=pl.BlockSpec((S, CO, PL), lambda g: (0, 0, 0)),
        scratch_shapes=[
            pltpu.VMEM((CO, 1), _F32),
            pltpu.VMEM((CO, 1), _F32),
            pltpu.SMEM((1,), _F32),
        ],
        out_shape=jax.ShapeDtypeStruct((S, CO, PL), _F32),
    )(x, x, x, m, w9)


def _down2(x8, m8, w):
    CO = w.shape[0]
    Nc = x8.shape[1]

    def body(x_ref, m_ref, w_ref, o_ref, mo_ref):
        mo = jnp.max(m_ref[...], axis=0, keepdims=True)
        y = jax.lax.dot(w_ref[...], x_ref[...], preferred_element_type=_F32)
        o_ref[...] = y * mo
        mo_ref[...] = mo

    return pl.pallas_call(
        body,
        out_shape=(jax.ShapeDtypeStruct((CO, Nc), _F32),
                   jax.ShapeDtypeStruct((1, Nc), _F32)))(x8, m8, w)


def _up2(f, m8, w8):
    CO = w8.shape[1]
    Nc = f.shape[1]

    def body(f_ref, m_ref, w_ref, o_ref):
        for a in range(8):
            y = jax.lax.dot(w_ref[a], f_ref[...], preferred_element_type=_F32)
            o_ref[a] = y * m_ref[a:a + 1, :]

    return pl.pallas_call(
        body, out_shape=jax.ShapeDtypeStruct((8, CO, Nc), _F32))(f, m8, w8)


def _to_planes(x, D, PL):
    C = x.shape[0]
    S = D + 2
    xp = jnp.pad(x, ((0, 0), (1, 1), (1, 1), (1, 1)))
    xp = jnp.transpose(xp.reshape(C, S, S * S), (1, 0, 2))
    return jnp.pad(xp, ((0, 0), (0, 0), (_PAD, PL - _PAD - S * S)))


def _from_planes(x, D):
    S = D + 2
    v = x[1:D + 1, :, _PAD:_PAD + S * S]
    v = jnp.transpose(v, (1, 0, 2)).reshape(-1, D, S, S)
    return v[:, :, 1:D + 1, 1:D + 1]


def _sub8(x):
    C, D = x.shape[0], x.shape[1]
    h = D // 2
    y = x.reshape(C, h, 2, h, 2, h, 2)
    y = jnp.transpose(y, (2, 4, 6, 0, 1, 3, 5))
    return y.reshape(8, C, h * h * h)


def _interleave8(y8, CO, h):
    y = y8.reshape(2, 2, 2, CO, h, h, h)
    y = jnp.transpose(y, (3, 4, 0, 5, 1, 6, 2))
    return y.reshape(CO, 2 * h, 2 * h, 2 * h)


def _w27(w):
    return jnp.transpose(w, (2, 3, 4, 0, 1)).reshape(27, w.shape[0], w.shape[1])


def _wdown(w):
    return jnp.transpose(w, (0, 2, 3, 4, 1)).reshape(w.shape[0], 8 * w.shape[1])


def _wup(w):
    return jnp.transpose(w[:, :, ::-1, ::-1, ::-1],
                         (2, 3, 4, 0, 1)).reshape(8, w.shape[0], w.shape[1])


def kernel(voxel_feats, voxel_mask, W_init, b_init, db0_w1, db0_w2, db1_w1,
           db1_w2, db2_w1, db2_w2, ds0_w, ds1_w, us0_w, us1_w, ub0_w1, ub0_w2,
           ub1_w1, ub1_w2, out0_w, out1_w, out2_w):
    vf = voxel_feats[0].astype(_F32)
    m0d = voxel_mask[0].astype(_F32)
    D0 = vf.shape[-1]
    D1, D2 = D0 // 2, D0 // 4
    PL0, PL1, PL2 = _pl_lanes(D0), _pl_lanes(D1), _pl_lanes(D2)

    vf_p = _to_planes(vf, D0, PL0)
    m0_p = _to_planes(m0d, D0, PL0)
    x = _conv3(vf_p, m0_p, _w27(W_init), D0, b=b_init)
    x = _conv3(x, m0_p, _w27(db0_w1), D0, inorm=True, lrelu=True)
    r0 = _conv3(x, m0_p, _w27(db0_w2), D0, lrelu=True)
    r0_d = _from_planes(r0, D0)

    x8 = _sub8(r0_d).reshape(8 * r0_d.shape[0], D1 ** 3)
    m8 = _sub8(m0d).reshape(8, D1 ** 3)
    xd, m1f = _down2(x8, m8, _wdown(ds0_w))
    m1d = m1f.reshape(1, D1, D1, D1)
    x = _to_planes(xd.reshape(-1, D1, D1, D1), D1, PL1)
    m1_p = _to_planes(m1d, D1, PL1)
    x = _conv3(x, m1_p, _w27(db1_w1), D1, inorm=True, lrelu=True)
    r1 = _conv3(x, m1_p, _w27(db1_w2), D1, lrelu=True)
    r1_d = _from_planes(r1, D1)

    x8 = _sub8(r1_d).reshape(8 * r1_d.shape[0], D2 ** 3)
    m8 = _sub8(m1d).reshape(8, D2 ** 3)
    xd, m2f = _down2(x8, m8, _wdown(ds1_w))
    m2d = m2f.reshape(1, D2, D2, D2)
    x = _to_planes(xd.reshape(-1, D2, D2, D2), D2, PL2)
    m2_p = _to_planes(m2d, D2, PL2)
    x = _conv3(x, m2_p, _w27(db2_w1), D2, inorm=True, lrelu=True)
    f0p = _conv3(x, m2_p, _w27(db2_w2), D2, lrelu=True)
    out0p = _conv3(f0p, m2_p, _w27(out0_w), D2)
    f0_d = _from_planes(f0p, D2)

    m1_8 = _sub8(m1d).reshape(8, D2 ** 3)
    y8 = _up2(f0_d.reshape(-1, D2 ** 3), m1_8, _wup(us0_w))
    xup = _interleave8(y8, us0_w.shape[0], D2)
    cat = jnp.concatenate([r1_d, xup], axis=0)
    x = _conv3(_to_planes(cat, D1, PL1), m1_p, _w27(ub0_w1), D1,
               inorm=True, lrelu=True)
    f1p = _conv3(x, m1_p, _w27(ub0_w2), D1, lrelu=True)
    out1p = _conv3(f1p, m1_p, _w27(out1_w), D1)
    f1_d = _from_planes(f1p, D1)

    m0_8 = _sub8(m0d).reshape(8, D1 ** 3)
    y8 = _up2(f1_d.reshape(-1, D1 ** 3), m0_8, _wup(us1_w))
    xup = _interleave8(y8, us1_w.shape[0], D1)
    cat = jnp.concatenate([r0_d, xup], axis=0)
    x = _conv3_stream(_to_planes(cat, D0, PL0), m0_p, _w27(ub1_w1), D0,
                      inorm=True, lrelu=True)
    f2p = _conv3(x, m0_p, _w27(ub1_w2), D0, lrelu=True)
    out2p = _conv3(f2p, m0_p, _w27(out2_w), D0)

    out0 = _from_planes(out0p, D2)[None]
    out1 = _from_planes(out1p, D1)[None]
    out2 = _from_planes(out2p, D0)[None]
    f0 = f0_d[None]
    f1 = _from_planes(f1p, D1)[None]
    f2 = _from_planes(f2p, D0)[None]
    return ((out0, out1, out2), (f0, f1, f2))


# speedup vs baseline: 1.3435x; 1.0577x over previous
import jax
import jax.numpy as jnp
from jax.experimental import pallas as pl
from jax.experimental.pallas import tpu as pltpu

_F32 = jnp.float32
_PAD = 128


def _pl_lanes(D):
    S = D + 2
    need = _PAD + S * S + S + 1
    return -(-need // 128) * 128


def _offs(S):
    return [_PAD + dy * S + dx for dy in (-1, 0, 1) for dx in (-1, 0, 1)]


def _merge_all(CI, S2):
    return 27 * CI * S2 * 4 <= 9 * 1024 * 1024


def _wgroups(w, CI, S2):
    CO = w.shape[1]
    if _merge_all(CI, S2):
        return jnp.transpose(w, (1, 0, 2)).reshape(1, CO, 27 * CI)
    return jnp.transpose(w.reshape(3, 9, CO, CI), (0, 2, 1, 3)).reshape(3, CO, 9 * CI)


def _conv_plane(vs, w_ref, offs, CI, S2, merged):
    if merged:
        xc = jnp.concatenate(
            [jax.lax.slice(v, (0, o), (CI, o + S2)) for v in vs for o in offs],
            axis=0)
        return jax.lax.dot(w_ref[0], xc, preferred_element_type=_F32)
    y = None
    for dz in range(3):
        xc = jnp.concatenate(
            [jax.lax.slice(vs[dz], (0, o), (CI, o + S2)) for o in offs], axis=0)
        t = jax.lax.dot(w_ref[dz], xc, preferred_element_type=_F32)
        y = t if y is None else y + t
    return y


def _chain(x, m, layers, D, head_w=None):
    S = D + 2
    S2 = S * S
    PL = x.shape[2]
    offs = _offs(S)
    L = len(layers)
    COs = [w.shape[1] for (w, _, _, _) in layers]
    CIs = [x.shape[1]] + COs[:-1]
    merged = [_merge_all(ci, S2) for ci in CIs]
    wgs = [_wgroups(w, ci, S2) for (w, _, _, _), ci in zip(layers, CIs)]
    biases = [b.reshape(-1, 1) if b is not None else None
              for (_, b, _, _) in layers]
    use_A = [((L - 1 - i) % 2) == 1 for i in range(L)]
    CA = max([COs[i] for i in range(L) if use_A[i]], default=1)
    if head_w is not None:
        h_merged = _merge_all(COs[-1], S2)
        h_wg = _wgroups(head_w, COs[-1], S2)

    def body(*refs):
        it = iter(refs)
        x_ref = next(it)
        m_ref = next(it)
        w_refs = [next(it) for _ in range(L)]
        b_refs = [next(it) if b is not None else None for b in biases]
        if head_w is not None:
            hw_ref = next(it)
        o_ref = next(it)
        if head_w is not None:
            h_ref = next(it)
        a_ref = next(it)

        def run_stage(i, src_ref):
            dst = a_ref if use_A[i] else o_ref
            CI_i, CO_i = CIs[i], COs[i]
            _, _, inorm, lrelu = layers[i]
            dst[0] = jnp.zeros((CO_i, PL), _F32)
            dst[S - 1] = jnp.zeros((CO_i, PL), _F32)

            def plane(g, carry):
                s1, s2, nn = carry
                vs = [src_ref[g + dz] for dz in range(3)]
                y = _conv_plane(vs, w_refs[i], offs, CI_i, S2, merged[i])
                if b_refs[i] is not None:
                    y = y + b_refs[i][...]
                mm = jax.lax.slice(m_ref[g + 1], (0, _PAD), (1, _PAD + S2))
                y = y * mm
                if lrelu and not inorm:
                    y = jnp.where(y >= 0, y, 0.2 * y)
                dst[g + 1] = jnp.pad(y, ((0, 0), (_PAD, PL - _PAD - S2)))
                if inorm:
                    s1 = s1 + jnp.sum(y, axis=1, keepdims=True)
                    s2 = s2 + jnp.sum(y * y, axis=1, keepdims=True)
                    nn = nn + jnp.sum(mm)
                return (s1, s2, nn)

            init = (jnp.zeros((CO_i, 1), _F32), jnp.zeros((CO_i, 1), _F32),
                    _F32(0))
            s1, s2, nn = jax.lax.fori_loop(0, D, plane, init)
            if inorm:
                n = jnp.maximum(nn, 1.0)
                mu = s1 / n
                var = s2 / n - mu * mu
                inv = jax.lax.rsqrt(var + 1e-5)

                def norm_fix(z, _):
                    v = (dst[z] - mu) * inv * m_ref[z]
                    if lrelu:
                        v = jnp.where(v >= 0, v, 0.2 * v)
                    dst[z] = v
                    return 0

                jax.lax.fori_loop(1, D + 1, norm_fix, 0)
            return dst

        src = x_ref
        for i in range(L):
            src = run_stage(i, src)

        if head_w is not None:
            h_ref[0] = jnp.zeros((1, PL), _F32)
            h_ref[S - 1] = jnp.zeros((1, PL), _F32)

            def hplane(g, _):
                vs = [o_ref[g + dz] for dz in range(3)]
                y = _conv_plane(vs, hw_ref, offs, COs[-1], S2, h_merged)
                mm = jax.lax.slice(m_ref[g + 1], (0, _PAD), (1, _PAD + S2))
                y = y * mm
                h_ref[g + 1] = jnp.pad(y, ((0, 0), (_PAD, PL - _PAD - S2)))
                return 0

            jax.lax.fori_loop(0, D, hplane, 0)

    args = [x, m] + wgs + [b for b in biases if b is not None]
    out_shapes = [jax.ShapeDtypeStruct((S, COs[-1], PL), _F32)]
    if head_w is not None:
        args = args + [h_wg]
        out_shapes.append(jax.ShapeDtypeStruct((S, 1, PL), _F32))
    res = pl.pallas_call(
        body,
        out_shape=tuple(out_shapes),
        scratch_shapes=[pltpu.VMEM((S, CA, PL), _F32)],
    )(*args)
    return res if head_w is not None else res[0]


def _conv3_stream2(xa, xb, m, w, D, inorm=False, lrelu=False):
    S = D + 2
    S2 = S * S
    PL = xa.shape[2]
    CA_, CB_ = xa.shape[1], xb.shape[1]
    CO = w.shape[1]
    offs = _offs(S)
    wa = _wgroups(w[:, :, :CA_], CA_, S2)
    wb = _wgroups(w[:, :, CA_:], CB_, S2)
    ma_, mb_ = _merge_all(CA_, S2), _merge_all(CB_, S2)

    def body(a0, a1, a2, b0, b1, b2, m_ref, wa_ref, wb_ref, o_ref,
             s1_ref, s2_ref, nn_ref):
        g = pl.program_id(0)

        @pl.when(g == 0)
        def _init():
            s1_ref[...] = jnp.zeros((CO, 1), _F32)
            s2_ref[...] = jnp.zeros((CO, 1), _F32)
            nn_ref[0] = _F32(0)
            o_ref[0] = jnp.zeros((CO, PL), _F32)
            o_ref[S - 1] = jnp.zeros((CO, PL), _F32)

        va = [a0[0], a1[0], a2[0]]
        vb = [b0[0], b1[0], b2[0]]
        y = (_conv_plane(va, wa_ref, offs, CA_, S2, ma_) +
             _conv_plane(vb, wb_ref, offs, CB_, S2, mb_))
        mm = jax.lax.slice(m_ref[g + 1], (0, _PAD), (1, _PAD + S2))
        y = y * mm
        if lrelu and not inorm:
            y = jnp.where(y >= 0, y, 0.2 * y)
        o_ref[g + 1] = jnp.pad(y, ((0, 0), (_PAD, PL - _PAD - S2)))
        if inorm:
            s1_ref[...] += jnp.sum(y, axis=1, keepdims=True)
            s2_ref[...] += jnp.sum(y * y, axis=1, keepdims=True)
            nn_ref[0] += jnp.sum(mm)

            @pl.when(g == D - 1)
            def _finish():
                n = jnp.maximum(nn_ref[0], 1.0)
                mu = s1_ref[...] / n
                var = s2_ref[...] / n - mu * mu
                inv = jax.lax.rsqrt(var + 1e-5)

                def norm(z, _):
                    v = (o_ref[z] - mu) * inv * m_ref[z]
                    if lrelu:
                        v = jnp.where(v >= 0, v, 0.2 * v)
                    o_ref[z] = v
                    return 0

                jax.lax.fori_loop(1, D + 1, norm, 0)

    pspec = lambda C: [pl.BlockSpec((1, C, PL), (lambda d: lambda g: (g + d, 0, 0))(d))
                       for d in range(3)]
    return pl.pallas_call(
        body,
        grid=(D,),
        in_specs=pspec(CA_) + pspec(CB_) + [
            pl.BlockSpec((S, 1, PL), lambda g: (0, 0, 0)),
            pl.BlockSpec(wa.shape, lambda g: (0, 0, 0)),
            pl.BlockSpec(wb.shape, lambda g: (0, 0, 0)),
        ],
        out_specs=pl.BlockSpec((S, CO, PL), lambda g: (0, 0, 0)),
        scratch_shapes=[
            pltpu.VMEM((CO, 1), _F32),
            pltpu.VMEM((CO, 1), _F32),
            pltpu.SMEM((1,), _F32),
        ],
        out_shape=jax.ShapeDtypeStruct((S, CO, PL), _F32),
    )(xa, xa, xa, xb, xb, xb, m, wa, wb)


def _down2(x8, m8, w):
    CO = w.shape[0]
    Nc = x8.shape[1]

    def body(x_ref, m_ref, w_ref, o_ref, mo_ref):
        mo = jnp.max(m_ref[...], axis=0, keepdims=True)
        y = jax.lax.dot(w_ref[...], x_ref[...], preferred_element_type=_F32)
        o_ref[...] = y * mo
        mo_ref[...] = mo

    return pl.pallas_call(
        body,
        out_shape=(jax.ShapeDtypeStruct((CO, Nc), _F32),
                   jax.ShapeDtypeStruct((1, Nc), _F32)))(x8, m8, w)


def _up2(f, m8, w8):
    CO = w8.shape[1]
    Nc = f.shape[1]

    def body(f_ref, m_ref, w_ref, o_ref):
        for a in range(8):
            y = jax.lax.dot(w_ref[a], f_ref[...], preferred_element_type=_F32)
            o_ref[a] = y * m_ref[a:a + 1, :]

    return pl.pallas_call(
        body, out_shape=jax.ShapeDtypeStruct((8, CO, Nc), _F32))(f, m8, w8)


def _to_planes(x, D, PL):
    C = x.shape[0]
    S = D + 2
    xp = jnp.pad(x, ((0, 0), (1, 1), (1, 1), (1, 1)))
    xp = jnp.transpose(xp.reshape(C, S, S * S), (1, 0, 2))
    return jnp.pad(xp, ((0, 0), (0, 0), (_PAD, PL - _PAD - S * S)))


def _from_planes(x, D):
    S = D + 2
    v = x[1:D + 1, :, _PAD:_PAD + S * S]
    v = jnp.transpose(v, (1, 0, 2)).reshape(-1, D, S, S)
    return v[:, :, 1:D + 1, 1:D + 1]


def _sub8(x):
    C, D = x.shape[0], x.shape[1]
    h = D // 2
    y = x.reshape(C, h, 2, h, 2, h, 2)
    y = jnp.transpose(y, (2, 4, 6, 0, 1, 3, 5))
    return y.reshape(8, C, h * h * h)


def _interleave8(y8, CO, h):
    y = y8.reshape(2, 2, 2, CO, h, h, h)
    y = jnp.transpose(y, (3, 4, 0, 5, 1, 6, 2))
    return y.reshape(CO, 2 * h, 2 * h, 2 * h)


def _w27(w):
    return jnp.transpose(w, (2, 3, 4, 0, 1)).reshape(27, w.shape[0], w.shape[1])


def _wdown(w):
    return jnp.transpose(w, (0, 2, 3, 4, 1)).reshape(w.shape[0], 8 * w.shape[1])


def _wup(w):
    return jnp.transpose(w[:, :, ::-1, ::-1, ::-1],
                         (2, 3, 4, 0, 1)).reshape(8, w.shape[0], w.shape[1])


def kernel(voxel_feats, voxel_mask, W_init, b_init, db0_w1, db0_w2, db1_w1,
           db1_w2, db2_w1, db2_w2, ds0_w, ds1_w, us0_w, us1_w, ub0_w1, ub0_w2,
           ub1_w1, ub1_w2, out0_w, out1_w, out2_w):
    vf = voxel_feats[0].astype(_F32)
    m0d = voxel_mask[0].astype(_F32)
    D0 = vf.shape[-1]
    D1, D2 = D0 // 2, D0 // 4
    PL0, PL1, PL2 = _pl_lanes(D0), _pl_lanes(D1), _pl_lanes(D2)

    vf_p = _to_planes(vf, D0, PL0)
    m0_p = _to_planes(m0d, D0, PL0)
    r0_p = _chain(vf_p, m0_p,
                  [(_w27(W_init), b_init, False, False),
                   (_w27(db0_w1), None, True, True),
                   (_w27(db0_w2), None, False, True)], D0)
    r0_d = _from_planes(r0_p, D0)

    x8 = _sub8(r0_d).reshape(8 * r0_d.shape[0], D1 ** 3)
    m8 = _sub8(m0d).reshape(8, D1 ** 3)
    xd, m1f = _down2(x8, m8, _wdown(ds0_w))
    m1d = m1f.reshape(1, D1, D1, D1)
    x1_p = _to_planes(xd.reshape(-1, D1, D1, D1), D1, PL1)
    m1_p = _to_planes(m1d, D1, PL1)
    r1_p = _chain(x1_p, m1_p,
                  [(_w27(db1_w1), None, True, True),
                   (_w27(db1_w2), None, False, True)], D1)
    r1_d = _from_planes(r1_p, D1)

    x8 = _sub8(r1_d).reshape(8 * r1_d.shape[0], D2 ** 3)
    m8 = _sub8(m1d).reshape(8, D2 ** 3)
    xd, m2f = _down2(x8, m8, _wdown(ds1_w))
    m2d = m2f.reshape(1, D2, D2, D2)
    x2_p = _to_planes(xd.reshape(-1, D2, D2, D2), D2, PL2)
    m2_p = _to_planes(m2d, D2, PL2)
    f0p, out0p = _chain(x2_p, m2_p,
                        [(_w27(db2_w1), None, True, True),
                         (_w27(db2_w2), None, False, True)], D2,
                        head_w=_w27(out0_w))
    f0_d = _from_planes(f0p, D2)

    m1_8 = _sub8(m1d).reshape(8, D2 ** 3)
    y8 = _up2(f0_d.reshape(-1, D2 ** 3), m1_8, _wup(us0_w))
    xup = _interleave8(y8, us0_w.shape[0], D2)
    cat = jnp.concatenate([r1_d, xup], axis=0)
    f1p, out1p = _chain(_to_planes(cat, D1, PL1), m1_p,
                        [(_w27(ub0_w1), None, True, True),
                         (_w27(ub0_w2), None, False, True)], D1,
                        head_w=_w27(out1_w))
    f1_d = _from_planes(f1p, D1)

    m0_8 = _sub8(m0d).reshape(8, D1 ** 3)
    y8 = _up2(f1_d.reshape(-1, D1 ** 3), m0_8, _wup(us1_w))
    xup_p = _to_planes(_interleave8(y8, us1_w.shape[0], D1), D0, PL0)
    x_p = _conv3_stream2(r0_p, xup_p, m0_p, _w27(ub1_w1), D0,
                         inorm=True, lrelu=True)
    f2p, out2p = _chain(x_p, m0_p,
                        [(_w27(ub1_w2), None, False, True)], D0,
                        head_w=_w27(out2_w))

    out0 = _from_planes(out0p, D2)[None]
    out1 = _from_planes(out1p, D1)[None]
    out2 = _from_planes(out2p, D0)[None]
    f0 = f0_d[None]
    f1 = _from_planes(f1p, D1)[None]
    f2 = _from_planes(f2p, D0)[None]
    return ((out0, out1, out2), (f0, f1, f2))


# speedup vs baseline: 1.4206x; 1.0574x over previous
import jax
import jax.numpy as jnp
from jax.experimental import pallas as pl
from jax.experimental.pallas import tpu as pltpu

_F32 = jnp.float32
_BF16 = jnp.float32
_PAD = 128


def _pl_lanes(D):
    S = D + 2
    need = _PAD + S * S + S + 1
    return -(-need // 128) * 128


def _offs(S):
    return [_PAD + dy * S + dx for dy in (-1, 0, 1) for dx in (-1, 0, 1)]


def _merge_all(CI, S2):
    return 27 * CI * S2 * 4 <= 9 * 1024 * 1024


def _wgroups(w, CI, S2):
    CO = w.shape[1]
    if _merge_all(CI, S2):
        wg = jnp.transpose(w, (1, 0, 2)).reshape(1, CO, 27 * CI)
    else:
        wg = jnp.transpose(w.reshape(3, 9, CO, CI), (0, 2, 1, 3)).reshape(3, CO, 9 * CI)
    return wg.astype(_BF16)


def _conv_plane(vs, w_ref, offs, CI, S2, merged):
    if merged:
        xc = jnp.concatenate(
            [jax.lax.slice(v, (0, o), (CI, o + S2)) for v in vs for o in offs],
            axis=0)
        return jax.lax.dot(w_ref[0], xc, preferred_element_type=_F32)
    y = None
    for dz in range(3):
        xc = jnp.concatenate(
            [jax.lax.slice(vs[dz], (0, o), (CI, o + S2)) for o in offs], axis=0)
        t = jax.lax.dot(w_ref[dz], xc, preferred_element_type=_F32)
        y = t if y is None else y + t
    return y


def _chain(x, m, layers, D, head_w=None, out_dtype=_BF16):
    S = D + 2
    S2 = S * S
    PL = x.shape[2]
    offs = _offs(S)
    L = len(layers)
    COs = [w.shape[1] for (w, _, _, _) in layers]
    CIs = [x.shape[1]] + COs[:-1]
    merged = [_merge_all(ci, S2) for ci in CIs]
    wgs = [_wgroups(w, ci, S2) for (w, _, _, _), ci in zip(layers, CIs)]
    biases = [b.reshape(-1, 1) if b is not None else None
              for (_, b, _, _) in layers]
    use_A = [((L - 1 - i) % 2) == 1 for i in range(L)]
    CA = max([COs[i] for i in range(L) if use_A[i]], default=1)
    if head_w is not None:
        h_merged = _merge_all(COs[-1], S2)
        h_wg = _wgroups(head_w, COs[-1], S2)

    def body(*refs):
        it = iter(refs)
        x_ref = next(it)
        m_ref = next(it)
        w_refs = [next(it) for _ in range(L)]
        b_refs = [next(it) if b is not None else None for b in biases]
        if head_w is not None:
            hw_ref = next(it)
        o_ref = next(it)
        if head_w is not None:
            h_ref = next(it)
        a_ref = next(it)

        def run_stage(i, src_ref):
            dst = a_ref if use_A[i] else o_ref
            CI_i, CO_i = CIs[i], COs[i]
            _, _, inorm, lrelu = layers[i]
            dst[0] = jnp.zeros((CO_i, PL), dst.dtype)
            dst[S - 1] = jnp.zeros((CO_i, PL), dst.dtype)

            def plane(g, carry):
                s1, s2, nn = carry
                vs = [src_ref[g + dz].astype(_BF16) for dz in range(3)]
                y = _conv_plane(vs, w_refs[i], offs, CI_i, S2, merged[i])
                if b_refs[i] is not None:
                    y = y + b_refs[i][...]
                mm = jax.lax.slice(m_ref[g + 1], (0, _PAD), (1, _PAD + S2))
                y = y * mm
                if lrelu and not inorm:
                    y = jnp.where(y >= 0, y, 0.2 * y)
                dst[g + 1] = jnp.pad(y, ((0, 0), (_PAD, PL - _PAD - S2))).astype(dst.dtype)
                if inorm:
                    s1 = s1 + jnp.sum(y, axis=1, keepdims=True)
                    s2 = s2 + jnp.sum(y * y, axis=1, keepdims=True)
                    nn = nn + jnp.sum(mm)
                return (s1, s2, nn)

            init = (jnp.zeros((CO_i, 1), _F32), jnp.zeros((CO_i, 1), _F32),
                    _F32(0))
            s1, s2, nn = jax.lax.fori_loop(0, D, plane, init)
            if inorm:
                n = jnp.maximum(nn, 1.0)
                mu = s1 / n
                var = s2 / n - mu * mu
                inv = jax.lax.rsqrt(var + 1e-5)

                def norm_fix(z, _):
                    v = (dst[z].astype(_F32) - mu) * inv * m_ref[z]
                    if lrelu:
                        v = jnp.where(v >= 0, v, 0.2 * v)
                    dst[z] = v.astype(dst.dtype)
                    return 0

                jax.lax.fori_loop(1, D + 1, norm_fix, 0)
            return dst

        src = x_ref
        for i in range(L):
            src = run_stage(i, src)

        if head_w is not None:
            h_ref[0] = jnp.zeros((1, PL), _F32)
            h_ref[S - 1] = jnp.zeros((1, PL), _F32)

            def hplane(g, _):
                vs = [o_ref[g + dz].astype(_BF16) for dz in range(3)]
                y = _conv_plane(vs, hw_ref, offs, COs[-1], S2, h_merged)
                mm = jax.lax.slice(m_ref[g + 1], (0, _PAD), (1, _PAD + S2))
                y = y * mm
                h_ref[g + 1] = jnp.pad(y, ((0, 0), (_PAD, PL - _PAD - S2)))
                return 0

            jax.lax.fori_loop(0, D, hplane, 0)

    args = [x, m] + wgs + [b for b in biases if b is not None]
    out_shapes = [jax.ShapeDtypeStruct((S, COs[-1], PL), out_dtype)]
    if head_w is not None:
        args = args + [h_wg]
        out_shapes.append(jax.ShapeDtypeStruct((S, 1, PL), _F32))
    res = pl.pallas_call(
        body,
        out_shape=tuple(out_shapes),
        scratch_shapes=[pltpu.VMEM((S, CA, PL), _BF16)],
    )(*args)
    return res if head_w is not None else res[0]


_ROW = 128
_LEAD = 256


def _fat_lanes(D):
    S = D + 2
    return _LEAD + S * _ROW + _LEAD


def _conv_plane_fat(v, w_ref, CI, S):
    SP = S * _ROW
    vq = jax.lax.slice(v, (0, _LEAD - 129), (CI, _LEAD - 129 + 256 + SP))
    vp = jax.lax.slice(v, (0, _LEAD - 127), (CI, _LEAD - 127 + 256 + SP))
    parts = []
    for dy in (-1, 0, 1):
        s = (dy + 1) * _ROW
        parts.append(jax.lax.slice(vq, (0, s), (CI, s + SP)))
        parts.append(jax.lax.slice(v, (0, _LEAD - 128 + s), (CI, _LEAD - 128 + s + SP)))
        parts.append(jax.lax.slice(vp, (0, s), (CI, s + SP)))
    return jnp.concatenate(parts, axis=0)


def _conv_plane_fat9(v, w_ref, CI, S):
    SP = S * _ROW
    vq = jax.lax.slice(v, (0, _LEAD - 129), (CI, _LEAD - 129 + 256 + SP))
    vp = jax.lax.slice(v, (0, _LEAD - 127), (CI, _LEAD - 127 + 256 + SP))
    y = None
    for j, dy in enumerate((-1, 0, 1)):
        s = (dy + 1) * _ROW
        xc = jnp.concatenate([
            jax.lax.slice(vq, (0, s), (CI, s + SP)),
            jax.lax.slice(v, (0, _LEAD - 128 + s), (CI, _LEAD - 128 + s + SP)),
            jax.lax.slice(vp, (0, s), (CI, s + SP))], axis=0)
        t = jax.lax.dot(w_ref[j], xc, preferred_element_type=_F32)
        y = t if y is None else y + t
    return y


def _wfat(w, CI):
    CO = w.shape[1]
    return jnp.transpose(w.reshape(3, 9, CO, CI), (0, 2, 1, 3)).reshape(
        3, CO, 9 * CI)


def _wfat9(w, CI):
    CO = w.shape[1]
    return jnp.transpose(w.reshape(3, 3, 3, CO, CI), (0, 1, 3, 2, 4)).reshape(
        3, 3, CO, 3 * CI)


def _conv_stream_fat(xs, m, w, D, bias=None, inorm=False, lrelu=False,
                     head_w=None, out_dtype=_F32):
    S = D + 2
    SP = S * _ROW
    PLf = xs[0].shape[2]
    CIs_ = [x.shape[1] for x in xs]
    CO = w.shape[1]
    offs_ci = [sum(CIs_[:j]) for j in range(len(CIs_))]
    wgs = []
    for j, CI_j in enumerate(CIs_):
        wj = w[:, :, offs_ci[j]:offs_ci[j] + CI_j]
        wgs.append(_wfat9(wj, CI_j) if CI_j > 16 else _wfat(wj, CI_j))
    if head_w is not None:
        hw = _wfat9(head_w, CO) if CO > 16 else _wfat(head_w, CO)

    def body(*refs):
        it = iter(refs)
        x_refs = [[next(it) for _ in range(3)] for _ in CIs_]
        m_ref = next(it)
        w_refs = [next(it) for _ in CIs_]
        b_ref = next(it) if bias is not None else None
        hw_ref = next(it) if head_w is not None else None
        o_ref = next(it)
        h_ref = next(it) if head_w is not None else None
        s1_ref, s2_ref, nn_ref = next(it), next(it), next(it)
        g = pl.program_id(0)

        @pl.when(g == 0)
        def _init():
            s1_ref[...] = jnp.zeros((CO, 1), _F32)
            s2_ref[...] = jnp.zeros((CO, 1), _F32)
            nn_ref[0] = _F32(0)
            o_ref[0] = jnp.zeros((CO, PLf), o_ref.dtype)
            o_ref[S - 1] = jnp.zeros((CO, PLf), o_ref.dtype)
            if head_w is not None:
                h_ref[0] = jnp.zeros((1, PLf), _F32)
                h_ref[S - 1] = jnp.zeros((1, PLf), _F32)

        def conv_from(v3, w_ref, CI_j):
            y = None
            for dz in range(3):
                v = v3[dz]
                if CI_j > 16:
                    t = _conv_plane_fat9(v, w_ref[dz], CI_j, S)
                else:
                    xc = _conv_plane_fat(v, None, CI_j, S)
                    t = jax.lax.dot(w_ref[dz], xc, preferred_element_type=_F32)
                y = t if y is None else y + t
            return y

        y = None
        for j in range(len(CIs_)):
            v3 = [x_refs[j][dz][0].astype(_F32) for dz in range(3)]
            t = conv_from(v3, w_refs[j], CIs_[j])
            y = t if y is None else y + t
        if bias is not None:
            y = y + b_ref[...]
        mm = jax.lax.slice(m_ref[g + 1], (0, _LEAD), (1, _LEAD + SP))
        y = y * mm
        if lrelu and not inorm:
            y = jnp.where(y >= 0, y, 0.2 * y)
        o_ref[g + 1] = jnp.pad(y, ((0, 0), (_LEAD, PLf - _LEAD - SP))).astype(o_ref.dtype)
        if inorm:
            s1_ref[...] += jnp.sum(y, axis=1, keepdims=True)
            s2_ref[...] += jnp.sum(y * y, axis=1, keepdims=True)
            nn_ref[0] += jnp.sum(mm)

        @pl.when(g == D - 1)
        def _finish():
            if inorm:
                n = jnp.maximum(nn_ref[0], 1.0)
                mu = s1_ref[...] / n
                var = s2_ref[...] / n - mu * mu
                inv = jax.lax.rsqrt(var + 1e-5)

                def norm(z, _):
                    v = (o_ref[z].astype(_F32) - mu) * inv * m_ref[z]
                    if lrelu:
                        v = jnp.where(v >= 0, v, 0.2 * v)
                    o_ref[z] = v.astype(o_ref.dtype)
                    return 0

                jax.lax.fori_loop(1, D + 1, norm, 0)
            if head_w is not None:
                def hplane(gz, _):
                    v3 = [o_ref[gz + dz].astype(_F32) for dz in range(3)]
                    if CO > 16:
                        yh = None
                        for dz in range(3):
                            t = _conv_plane_fat9(v3[dz], hw_ref[dz], CO, S)
                            yh = t if yh is None else yh + t
                    else:
                        yh = None
                        for dz in range(3):
                            xc = _conv_plane_fat(v3[dz], None, CO, S)
                            t = jax.lax.dot(hw_ref[dz], xc,
                                            preferred_element_type=_F32)
                            yh = t if yh is None else yh + t
                    mmh = jax.lax.slice(m_ref[gz + 1], (0, _LEAD), (1, _LEAD + SP))
                    yh = yh * mmh
                    h_ref[gz + 1] = jnp.pad(
                        yh, ((0, 0), (_LEAD, PLf - _LEAD - SP)))
                    return 0

                jax.lax.fori_loop(0, D, hplane, 0)

    in_specs = []
    args = []
    for j, x in enumerate(xs):
        for d in range(3):
            in_specs.append(pl.BlockSpec((1, CIs_[j], PLf),
                                         (lambda dd: lambda g: (g + dd, 0, 0))(d)))
            args.append(x)
    in_specs.append(pl.BlockSpec((S, 1, PLf), lambda g: (0, 0, 0)))
    args.append(m)
    for wg in wgs:
        in_specs.append(pl.BlockSpec(
            wg.shape, (lambda nd: lambda g: (0,) * nd)(len(wg.shape))))
        args.append(wg)
    if bias is not None:
        b2 = bias.reshape(CO, 1)
        in_specs.append(pl.BlockSpec(b2.shape, lambda g: (0, 0)))
        args.append(b2)
    if head_w is not None:
        in_specs.append(pl.BlockSpec(hw.shape, lambda g: tuple(0 for _ in hw.shape)))
        args.append(hw)
    out_shapes = [jax.ShapeDtypeStruct((S, CO, PLf), out_dtype)]
    out_specs = [pl.BlockSpec((S, CO, PLf), lambda g: (0, 0, 0))]
    if head_w is not None:
        out_shapes.append(jax.ShapeDtypeStruct((S, 1, PLf), _F32))
        out_specs.append(pl.BlockSpec((S, 1, PLf), lambda g: (0, 0, 0)))
    res = pl.pallas_call(
        body,
        grid=(D,),
        in_specs=in_specs,
        out_specs=out_specs,
        scratch_shapes=[
            pltpu.VMEM((CO, 1), _F32),
            pltpu.VMEM((CO, 1), _F32),
            pltpu.SMEM((1,), _F32),
        ],
        out_shape=tuple(out_shapes),
    )(*args)
    return res if head_w is not None else res[0]


def _to_planes_fat(x, D):
    C = x.shape[0]
    S = D + 2
    PLf = _fat_lanes(D)
    xp = jnp.pad(x, ((0, 0), (1, 1), (1, 1), (1, _ROW - D - 1)))
    xp = jnp.transpose(xp.reshape(C, S, S * _ROW), (1, 0, 2))
    return jnp.pad(xp, ((0, 0), (0, 0), (_LEAD, _LEAD)))


def _from_planes_fat(x, D):
    S = D + 2
    v = x[1:D + 1, :, _LEAD:_LEAD + S * _ROW]
    v = jnp.transpose(v, (1, 0, 2)).reshape(-1, D, S, _ROW)
    return v[:, :, 1:D + 1, 1:D + 1]


def _conv3_stream2(xa, xb, m, w, D, inorm=False, lrelu=False):
    S = D + 2
    S2 = S * S
    PL = xa.shape[2]
    CA_, CB_ = xa.shape[1], xb.shape[1]
    CO = w.shape[1]
    offs = _offs(S)
    wa = _wgroups(w[:, :, :CA_], CA_, S2)
    wb = _wgroups(w[:, :, CA_:], CB_, S2)
    ma_, mb_ = _merge_all(CA_, S2), _merge_all(CB_, S2)

    def body(a0, a1, a2, b0, b1, b2, m_ref, wa_ref, wb_ref, o_ref,
             s1_ref, s2_ref, nn_ref):
        g = pl.program_id(0)

        @pl.when(g == 0)
        def _init():
            s1_ref[...] = jnp.zeros((CO, 1), _F32)
            s2_ref[...] = jnp.zeros((CO, 1), _F32)
            nn_ref[0] = _F32(0)
            o_ref[0] = jnp.zeros((CO, PL), _BF16)
            o_ref[S - 1] = jnp.zeros((CO, PL), _BF16)

        va = [a0[0], a1[0], a2[0]]
        vb = [b0[0], b1[0], b2[0]]
        y = (_conv_plane(va, wa_ref, offs, CA_, S2, ma_) +
             _conv_plane(vb, wb_ref, offs, CB_, S2, mb_))
        mm = jax.lax.slice(m_ref[g + 1], (0, _PAD), (1, _PAD + S2))
        y = y * mm
        if lrelu and not inorm:
            y = jnp.where(y >= 0, y, 0.2 * y)
        o_ref[g + 1] = jnp.pad(y, ((0, 0), (_PAD, PL - _PAD - S2))).astype(_BF16)
        if inorm:
            s1_ref[...] += jnp.sum(y, axis=1, keepdims=True)
            s2_ref[...] += jnp.sum(y * y, axis=1, keepdims=True)
            nn_ref[0] += jnp.sum(mm)

            @pl.when(g == D - 1)
            def _finish():
                n = jnp.maximum(nn_ref[0], 1.0)
                mu = s1_ref[...] / n
                var = s2_ref[...] / n - mu * mu
                inv = jax.lax.rsqrt(var + 1e-5)

                def norm(z, _):
                    v = (o_ref[z].astype(_F32) - mu) * inv * m_ref[z]
                    if lrelu:
                        v = jnp.where(v >= 0, v, 0.2 * v)
                    o_ref[z] = v.astype(_BF16)
                    return 0

                jax.lax.fori_loop(1, D + 1, norm, 0)

    pspec = lambda C: [pl.BlockSpec((1, C, PL), (lambda d: lambda g: (g + d, 0, 0))(d))
                       for d in range(3)]
    return pl.pallas_call(
        body,
        grid=(D,),
        in_specs=pspec(CA_) + pspec(CB_) + [
            pl.BlockSpec((S, 1, PL), lambda g: (0, 0, 0)),
            pl.BlockSpec(wa.shape, lambda g: (0, 0, 0)),
            pl.BlockSpec(wb.shape, lambda g: (0, 0, 0)),
        ],
        out_specs=pl.BlockSpec((S, CO, PL), lambda g: (0, 0, 0)),
        scratch_shapes=[
            pltpu.VMEM((CO, 1), _F32),
            pltpu.VMEM((CO, 1), _F32),
            pltpu.SMEM((1,), _F32),
        ],
        out_shape=jax.ShapeDtypeStruct((S, CO, PL), _BF16),
    )(xa, xa, xa, xb, xb, xb, m, wa, wb)


def _down2(x8, m8, w):
    CO = w.shape[0]
    Nc = x8.shape[1]

    def body(x_ref, m_ref, w_ref, o_ref, mo_ref):
        mo = jnp.max(m_ref[...], axis=0, keepdims=True)
        y = jax.lax.dot(w_ref[...], x_ref[...], preferred_element_type=_F32)
        o_ref[...] = (y * mo).astype(_BF16)
        mo_ref[...] = mo

    return pl.pallas_call(
        body,
        out_shape=(jax.ShapeDtypeStruct((CO, Nc), _BF16),
                   jax.ShapeDtypeStruct((1, Nc), _F32)))(x8, m8, w)


def _up2(f, m8, w8):
    CO = w8.shape[1]
    Nc = f.shape[1]

    def body(f_ref, m_ref, w_ref, o_ref):
        for a in range(8):
            y = jax.lax.dot(w_ref[a], f_ref[...], preferred_element_type=_F32)
            o_ref[a] = (y * m_ref[a:a + 1, :]).astype(_BF16)

    return pl.pallas_call(
        body, out_shape=jax.ShapeDtypeStruct((8, CO, Nc), _BF16))(f, m8, w8)


def _to_planes(x, D, PL):
    C = x.shape[0]
    S = D + 2
    xp = jnp.pad(x, ((0, 0), (1, 1), (1, 1), (1, 1)))
    xp = jnp.transpose(xp.reshape(C, S, S * S), (1, 0, 2))
    return jnp.pad(xp, ((0, 0), (0, 0), (_PAD, PL - _PAD - S * S)))


def _from_planes(x, D):
    S = D + 2
    v = x[1:D + 1, :, _PAD:_PAD + S * S]
    v = jnp.transpose(v, (1, 0, 2)).reshape(-1, D, S, S)
    return v[:, :, 1:D + 1, 1:D + 1]


def _sub8(x):
    C, D = x.shape[0], x.shape[1]
    h = D // 2
    y = x.reshape(C, h, 2, h, 2, h, 2)
    y = jnp.transpose(y, (2, 4, 6, 0, 1, 3, 5))
    return y.reshape(8, C, h * h * h)


def _interleave8(y8, CO, h):
    y = y8.reshape(2, 2, 2, CO, h, h, h)
    y = jnp.transpose(y, (3, 4, 0, 5, 1, 6, 2))
    return y.reshape(CO, 2 * h, 2 * h, 2 * h)


def _w27(w):
    return jnp.transpose(w, (2, 3, 4, 0, 1)).reshape(27, w.shape[0], w.shape[1])


def _wdown(w):
    return jnp.transpose(w, (0, 2, 3, 4, 1)).reshape(
        w.shape[0], 8 * w.shape[1]).astype(_BF16)


def _wup(w):
    return jnp.transpose(w[:, :, ::-1, ::-1, ::-1],
                         (2, 3, 4, 0, 1)).reshape(
        8, w.shape[0], w.shape[1]).astype(_BF16)


def kernel(voxel_feats, voxel_mask, W_init, b_init, db0_w1, db0_w2, db1_w1,
           db1_w2, db2_w1, db2_w2, ds0_w, ds1_w, us0_w, us1_w, ub0_w1, ub0_w2,
           ub1_w1, ub1_w2, out0_w, out1_w, out2_w):
    vf = voxel_feats[0].astype(_BF16)
    m0d = voxel_mask[0].astype(_F32)
    D0 = vf.shape[-1]
    D1, D2 = D0 // 2, D0 // 4
    PL0, PL1, PL2 = _pl_lanes(D0), _pl_lanes(D1), _pl_lanes(D2)

    vf_p = _to_planes_fat(vf, D0)
    m0_p = _to_planes_fat(m0d, D0)
    x = _conv_stream_fat([vf_p], m0_p, _w27(W_init), D0, bias=b_init)
    x = _conv_stream_fat([x], m0_p, _w27(db0_w1), D0, inorm=True, lrelu=True)
    r0_p = _conv_stream_fat([x], m0_p, _w27(db0_w2), D0, lrelu=True)
    r0_d = _from_planes_fat(r0_p, D0)

    x8 = _sub8(r0_d).reshape(8 * r0_d.shape[0], D1 ** 3)
    m8 = _sub8(m0d).reshape(8, D1 ** 3)
    xd, m1f = _down2(x8, m8, _wdown(ds0_w))
    m1d = m1f.reshape(1, D1, D1, D1)
    x1_p = _to_planes(xd.reshape(-1, D1, D1, D1), D1, PL1)
    m1_p = _to_planes(m1d, D1, PL1)
    r1_p = _chain(x1_p, m1_p,
                  [(_w27(db1_w1), None, True, True),
                   (_w27(db1_w2), None, False, True)], D1)
    r1_d = _from_planes(r1_p, D1)

    x8 = _sub8(r1_d).reshape(8 * r1_d.shape[0], D2 ** 3)
    m8 = _sub8(m1d).reshape(8, D2 ** 3)
    xd, m2f = _down2(x8, m8, _wdown(ds1_w))
    m2d = m2f.reshape(1, D2, D2, D2)
    x2_p = _to_planes(xd.reshape(-1, D2, D2, D2), D2, PL2)
    m2_p = _to_planes(m2d, D2, PL2)
    f0p, out0p = _chain(x2_p, m2_p,
                        [(_w27(db2_w1), None, True, True),
                         (_w27(db2_w2), None, False, True)], D2,
                        head_w=_w27(out0_w), out_dtype=_F32)
    f0_d = _from_planes(f0p, D2)

    m1_8 = _sub8(m1d).reshape(8, D2 ** 3)
    y8 = _up2(f0_d.astype(_BF16).reshape(-1, D2 ** 3), m1_8, _wup(us0_w))
    xup = _interleave8(y8, us0_w.shape[0], D2)
    cat = jnp.concatenate([r1_d, xup], axis=0)
    f1p, out1p = _chain(_to_planes(cat, D1, PL1), m1_p,
                        [(_w27(ub0_w1), None, True, True),
                         (_w27(ub0_w2), None, False, True)], D1,
                        head_w=_w27(out1_w), out_dtype=_F32)
    f1_d = _from_planes(f1p, D1)

    m0_8 = _sub8(m0d).reshape(8, D1 ** 3)
    y8 = _up2(f1_d.astype(_BF16).reshape(-1, D1 ** 3), m0_8, _wup(us1_w))
    xup_p = _to_planes_fat(_interleave8(y8, us1_w.shape[0], D1), D0)
    x_p = _conv_stream_fat([r0_p, xup_p], m0_p, _w27(ub1_w1), D0,
                           inorm=True, lrelu=True, out_dtype=jnp.bfloat16)
    f2p, out2p = _conv_stream_fat([x_p], m0_p, _w27(ub1_w2), D0,
                                  lrelu=True, head_w=_w27(out2_w))

    out0 = _from_planes(out0p, D2)[None]
    out1 = _from_planes(out1p, D1)[None]
    out2 = _from_planes_fat(out2p, D0)[None]
    f0 = f0_d[None]
    f1 = _from_planes(f1p, D1)[None]
    f2 = _from_planes_fat(f2p, D0)[None]
    return ((out0, out1, out2), (f0, f1, f2))


# speedup vs baseline: 1.6276x; 1.1457x over previous
import jax
import jax.numpy as jnp
from jax.experimental import pallas as pl
from jax.experimental.pallas import tpu as pltpu

_F32 = jnp.float32
_PAD = 128
_ROW = 128
_LEAD = 256


def _pl_lanes(D):
    S = D + 2
    need = _PAD + S * S + S + 1
    return -(-need // 128) * 128


def _fat_lanes(D):
    S = D + 2
    return _LEAD + S * _ROW + _LEAD


def _wslab(w):
    CO, CI = w.shape[1], w.shape[2]
    return jnp.transpose(w.reshape(3, 3, 3, CO, CI),
                         (1, 2, 3, 0, 4)).reshape(9, CO, 3 * CI)


def _conv_slab_slim(V, w_ref, C3, S, S2):
    y = None
    for q, (dy, dx) in enumerate([(a, b) for a in (-1, 0, 1) for b in (-1, 0, 1)]):
        o = _PAD + dy * S + dx
        xc = jax.lax.slice(V, (0, o), (C3, o + S2))
        t = jax.lax.dot(w_ref[q], xc, preferred_element_type=_F32)
        y = t if y is None else y + t
    return y


def _conv_slab_fat(V, w_ref, C3, S):
    SP = S * _ROW
    vq = jax.lax.slice(V, (0, _LEAD - 129), (C3, _LEAD - 129 + 256 + SP))
    vp = jax.lax.slice(V, (0, _LEAD - 127), (C3, _LEAD - 127 + 256 + SP))
    y = None
    q = 0
    for dy in (-1, 0, 1):
        s = (dy + 1) * _ROW
        for src, s0 in ((vq, s), (V, _LEAD - 128 + s), (vp, s)):
            xc = jax.lax.slice(src, (0, s0), (C3, s0 + SP))
            t = jax.lax.dot(w_ref[q], xc, preferred_element_type=_F32)
            y = t if y is None else y + t
            q += 1
    return y


def _chain(x, m, layers, D, head_w=None, out_dtype=_F32):
    S = D + 2
    S2 = S * S
    PL = x.shape[2]
    L = len(layers)
    COs = [w.shape[1] for (w, _, _, _) in layers]
    CIs = [x.shape[1]] + COs[:-1]
    wgs = [_wslab(w) for (w, _, _, _) in layers]
    biases = [b.reshape(-1, 1) if b is not None else None
              for (_, b, _, _) in layers]
    use_A = [((L - 1 - i) % 2) == 1 for i in range(L)]
    CA = max([COs[i] for i in range(L) if use_A[i]], default=1)
    if head_w is not None:
        h_wg = _wslab(head_w)

    def body(*refs):
        it = iter(refs)
        x_ref = next(it)
        m_ref = next(it)
        w_refs = [next(it) for _ in range(L)]
        b_refs = [next(it) if b is not None else None for b in biases]
        if head_w is not None:
            hw_ref = next(it)
        o_ref = next(it)
        if head_w is not None:
            h_ref = next(it)
        a_ref = next(it)

        def run_stage(i, src_ref):
            dst = a_ref if use_A[i] else o_ref
            CI_i, CO_i = CIs[i], COs[i]
            _, _, inorm, lrelu = layers[i]
            dst[0] = jnp.zeros((CO_i, PL), dst.dtype)
            dst[S - 1] = jnp.zeros((CO_i, PL), dst.dtype)

            def plane(g, carry):
                s1, s2, nn = carry
                V = src_ref[pl.ds(g, 3)].reshape(3 * CI_i, PL)
                y = _conv_slab_slim(V, w_refs[i], 3 * CI_i, S, S2)
                if b_refs[i] is not None:
                    y = y + b_refs[i][...]
                mm = jax.lax.slice(m_ref[g + 1], (0, _PAD), (1, _PAD + S2))
                y = y * mm
                if lrelu and not inorm:
                    y = jnp.where(y >= 0, y, 0.2 * y)
                dst[g + 1] = jnp.pad(y, ((0, 0), (_PAD, PL - _PAD - S2)))
                if inorm:
                    s1 = s1 + jnp.sum(y, axis=1, keepdims=True)
                    s2 = s2 + jnp.sum(y * y, axis=1, keepdims=True)
                    nn = nn + jnp.sum(mm)
                return (s1, s2, nn)

            init = (jnp.zeros((CO_i, 1), _F32), jnp.zeros((CO_i, 1), _F32),
                    _F32(0))
            s1, s2, nn = jax.lax.fori_loop(0, D, plane, init)
            if inorm:
                n = jnp.maximum(nn, 1.0)
                mu = s1 / n
                var = s2 / n - mu * mu
                inv = jax.lax.rsqrt(var + 1e-5)

                def norm(z, _):
                    v = (dst[z] - mu) * inv * m_ref[z]
                    if lrelu:
                        v = jnp.where(v >= 0, v, 0.2 * v)
                    dst[z] = v
                    return 0

                jax.lax.fori_loop(1, D + 1, norm, 0)
            return dst

        src = x_ref
        for i in range(L):
            src = run_stage(i, src)

        if head_w is not None:
            h_ref[0] = jnp.zeros((1, PL), _F32)
            h_ref[S - 1] = jnp.zeros((1, PL), _F32)

            def hplane(g, _):
                V = o_ref[pl.ds(g, 3)].reshape(3 * COs[-1], PL)
                y = _conv_slab_slim(V, hw_ref, 3 * COs[-1], S, S2)
                mm = jax.lax.slice(m_ref[g + 1], (0, _PAD), (1, _PAD + S2))
                y = y * mm
                h_ref[g + 1] = jnp.pad(y, ((0, 0), (_PAD, PL - _PAD - S2)))
                return 0

            jax.lax.fori_loop(0, D, hplane, 0)

    args = [x, m] + wgs + [b for b in biases if b is not None]
    out_shapes = [jax.ShapeDtypeStruct((S, COs[-1], PL), out_dtype)]
    if head_w is not None:
        args = args + [h_wg]
        out_shapes.append(jax.ShapeDtypeStruct((S, 1, PL), _F32))
    res = pl.pallas_call(
        body,
        out_shape=tuple(out_shapes),
        scratch_shapes=[pltpu.VMEM((S, CA, PL), _F32)],
    )(*args)
    return res if head_w is not None else res[0]


def _conv_stream_fat(xs, m, w, D, bias=None, inorm=False, lrelu=False,
                     head_w=None, out_dtype=_F32):
    S = D + 2
    SP = S * _ROW
    PLf = xs[0].shape[2]
    CIs_ = [x.shape[1] for x in xs]
    CO = w.shape[1]
    off_ci = [sum(CIs_[:j]) for j in range(len(CIs_))]
    wgs = [_wslab(w[:, :, off_ci[j]:off_ci[j] + CIs_[j]]) for j in range(len(CIs_))]
    if head_w is not None:
        hw = _wslab(head_w)

    def body(*refs):
        it = iter(refs)
        x_refs = [[next(it) for _ in range(3)] for _ in CIs_]
        m_ref = next(it)
        w_refs = [next(it) for _ in CIs_]
        b_ref = next(it) if bias is not None else None
        hw_ref = next(it) if head_w is not None else None
        o_ref = next(it)
        h_ref = next(it) if head_w is not None else None
        s1_ref, s2_ref, nn_ref = next(it), next(it), next(it)
        g = pl.program_id(0)

        @pl.when(g == 0)
        def _init():
            s1_ref[...] = jnp.zeros((CO, 1), _F32)
            s2_ref[...] = jnp.zeros((CO, 1), _F32)
            nn_ref[0] = _F32(0)
            o_ref[0] = jnp.zeros((CO, PLf), o_ref.dtype)
            o_ref[S - 1] = jnp.zeros((CO, PLf), o_ref.dtype)
            if head_w is not None:
                h_ref[0] = jnp.zeros((1, PLf), _F32)
                h_ref[S - 1] = jnp.zeros((1, PLf), _F32)

        y = None
        for j in range(len(CIs_)):
            V = jnp.concatenate(
                [x_refs[j][dz][0].astype(_F32) for dz in range(3)], axis=0)
            t = _conv_slab_fat(V, w_refs[j], 3 * CIs_[j], S)
            y = t if y is None else y + t
        if bias is not None:
            y = y + b_ref[...]
        mm = jax.lax.slice(m_ref[g + 1], (0, _LEAD), (1, _LEAD + SP))
        y = y * mm
        if lrelu and not inorm:
            y = jnp.where(y >= 0, y, 0.2 * y)
        o_ref[g + 1] = jnp.pad(y, ((0, 0), (_LEAD, PLf - _LEAD - SP))).astype(o_ref.dtype)
        if inorm:
            s1_ref[...] += jnp.sum(y, axis=1, keepdims=True)
            s2_ref[...] += jnp.sum(y * y, axis=1, keepdims=True)
            nn_ref[0] += jnp.sum(mm)

        @pl.when(g == D - 1)
        def _finish():
            if inorm:
                n = jnp.maximum(nn_ref[0], 1.0)
                mu = s1_ref[...] / n
                var = s2_ref[...] / n - mu * mu
                inv = jax.lax.rsqrt(var + 1e-5)

                def norm(z, _):
                    v = (o_ref[z].astype(_F32) - mu) * inv * m_ref[z]
                    if lrelu:
                        v = jnp.where(v >= 0, v, 0.2 * v)
                    o_ref[z] = v.astype(o_ref.dtype)
                    return 0

                jax.lax.fori_loop(1, D + 1, norm, 0)
            if head_w is not None:
                def hplane(gz, _):
                    V = o_ref[pl.ds(gz, 3)].astype(_F32).reshape(3 * CO, PLf)
                    yh = _conv_slab_fat(V, hw_ref, 3 * CO, S)
                    mmh = jax.lax.slice(m_ref[gz + 1], (0, _LEAD), (1, _LEAD + SP))
                    yh = yh * mmh
                    h_ref[gz + 1] = jnp.pad(
                        yh, ((0, 0), (_LEAD, PLf - _LEAD - SP)))
                    return 0

                jax.lax.fori_loop(0, D, hplane, 0)

    in_specs = []
    args = []
    for j, x in enumerate(xs):
        for d in range(3):
            in_specs.append(pl.BlockSpec((1, CIs_[j], PLf),
                                         (lambda dd: lambda g: (g + dd, 0, 0))(d)))
            args.append(x)
    in_specs.append(pl.BlockSpec((S, 1, PLf), lambda g: (0, 0, 0)))
    args.append(m)
    for wg in wgs:
        in_specs.append(pl.BlockSpec(wg.shape, lambda g: (0, 0, 0)))
        args.append(wg)
    if bias is not None:
        b2 = bias.reshape(CO, 1)
        in_specs.append(pl.BlockSpec(b2.shape, lambda g: (0, 0)))
        args.append(b2)
    if head_w is not None:
        in_specs.append(pl.BlockSpec(hw.shape, lambda g: (0, 0, 0)))
        args.append(hw)
    out_shapes = [jax.ShapeDtypeStruct((S, CO, PLf), out_dtype)]
    out_specs = [pl.BlockSpec((S, CO, PLf), lambda g: (0, 0, 0))]
    if head_w is not None:
        out_shapes.append(jax.ShapeDtypeStruct((S, 1, PLf), _F32))
        out_specs.append(pl.BlockSpec((S, 1, PLf), lambda g: (0, 0, 0)))
    res = pl.pallas_call(
        body,
        grid=(D,),
        in_specs=in_specs,
        out_specs=out_specs,
        scratch_shapes=[
            pltpu.VMEM((CO, 1), _F32),
            pltpu.VMEM((CO, 1), _F32),
            pltpu.SMEM((1,), _F32),
        ],
        out_shape=tuple(out_shapes),
    )(*args)
    return res if head_w is not None else res[0]


def _down2(x8, m8, w):
    CO = w.shape[0]
    Nc = x8.shape[1]

    def body(x_ref, m_ref, w_ref, o_ref, mo_ref):
        mo = jnp.max(m_ref[...], axis=0, keepdims=True)
        y = jax.lax.dot(w_ref[...], x_ref[...], preferred_element_type=_F32)
        o_ref[...] = y * mo
        mo_ref[...] = mo

    return pl.pallas_call(
        body,
        out_shape=(jax.ShapeDtypeStruct((CO, Nc), _F32),
                   jax.ShapeDtypeStruct((1, Nc), _F32)))(x8, m8, w)


def _up2(f, m8, w8):
    CO = w8.shape[1]
    Nc = f.shape[1]

    def body(f_ref, m_ref, w_ref, o_ref):
        for a in range(8):
            y = jax.lax.dot(w_ref[a], f_ref[...], preferred_element_type=_F32)
            o_ref[a] = y * m_ref[a:a + 1, :]

    return pl.pallas_call(
        body, out_shape=jax.ShapeDtypeStruct((8, CO, Nc), _F32))(f, m8, w8)


def _to_planes(x, D, PL):
    C = x.shape[0]
    S = D + 2
    xp = jnp.pad(x, ((0, 0), (1, 1), (1, 1), (1, 1)))
    xp = jnp.transpose(xp.reshape(C, S, S * S), (1, 0, 2))
    return jnp.pad(xp, ((0, 0), (0, 0), (_PAD, PL - _PAD - S * S)))


def _from_planes(x, D):
    S = D + 2
    v = x[1:D + 1, :, _PAD:_PAD + S * S]
    v = jnp.transpose(v, (1, 0, 2)).reshape(-1, D, S, S)
    return v[:, :, 1:D + 1, 1:D + 1]


def _to_planes_fat(x, D):
    C = x.shape[0]
    S = D + 2
    xp = jnp.pad(x, ((0, 0), (1, 1), (1, 1), (1, _ROW - D - 1)))
    xp = jnp.transpose(xp.reshape(C, S, S * _ROW), (1, 0, 2))
    return jnp.pad(xp, ((0, 0), (0, 0), (_LEAD, _LEAD)))


def _from_planes_fat(x, D):
    S = D + 2
    v = x[1:D + 1, :, _LEAD:_LEAD + S * _ROW]
    v = jnp.transpose(v, (1, 0, 2)).reshape(-1, D, S, _ROW)
    return v[:, :, 1:D + 1, 1:D + 1]


def _sub8(x):
    C, D = x.shape[0], x.shape[1]
    h = D // 2
    y = x.reshape(C, h, 2, h, 2, h, 2)
    y = jnp.transpose(y, (2, 4, 6, 0, 1, 3, 5))
    return y.reshape(8, C, h * h * h)


def _interleave8(y8, CO, h):
    y = y8.reshape(2, 2, 2, CO, h, h, h)
    y = jnp.transpose(y, (3, 4, 0, 5, 1, 6, 2))
    return y.reshape(CO, 2 * h, 2 * h, 2 * h)


def _w27(w):
    return jnp.transpose(w, (2, 3, 4, 0, 1)).reshape(27, w.shape[0], w.shape[1])


def _wdown(w):
    return jnp.transpose(w, (0, 2, 3, 4, 1)).reshape(w.shape[0], 8 * w.shape[1])


def _wup(w):
    return jnp.transpose(w[:, :, ::-1, ::-1, ::-1],
                         (2, 3, 4, 0, 1)).reshape(8, w.shape[0], w.shape[1])


def kernel(voxel_feats, voxel_mask, W_init, b_init, db0_w1, db0_w2, db1_w1,
           db1_w2, db2_w1, db2_w2, ds0_w, ds1_w, us0_w, us1_w, ub0_w1, ub0_w2,
           ub1_w1, ub1_w2, out0_w, out1_w, out2_w):
    vf = voxel_feats[0].astype(_F32)
    m0d = voxel_mask[0].astype(_F32)
    D0 = vf.shape[-1]
    D1, D2 = D0 // 2, D0 // 4
    PL1, PL2 = _pl_lanes(D1), _pl_lanes(D2)

    vf_p = _to_planes_fat(vf, D0)
    m0_p = _to_planes_fat(m0d, D0)
    x = _conv_stream_fat([vf_p], m0_p, _w27(W_init), D0, bias=b_init)
    x = _conv_stream_fat([x], m0_p, _w27(db0_w1), D0, inorm=True, lrelu=True)
    r0_p = _conv_stream_fat([x], m0_p, _w27(db0_w2), D0, lrelu=True)
    r0_d = _from_planes_fat(r0_p, D0)

    x8 = _sub8(r0_d).reshape(8 * r0_d.shape[0], D1 ** 3)
    m8 = _sub8(m0d).reshape(8, D1 ** 3)
    xd, m1f = _down2(x8, m8, _wdown(ds0_w))
    m1d = m1f.reshape(1, D1, D1, D1)
    x1_p = _to_planes(xd.reshape(-1, D1, D1, D1), D1, PL1)
    m1_p = _to_planes(m1d, D1, PL1)
    r1_p = _chain(x1_p, m1_p,
                  [(_w27(db1_w1), None, True, True),
                   (_w27(db1_w2), None, False, True)], D1)
    r1_d = _from_planes(r1_p, D1)

    x8 = _sub8(r1_d).reshape(8 * r1_d.shape[0], D2 ** 3)
    m8 = _sub8(m1d).reshape(8, D2 ** 3)
    xd, m2f = _down2(x8, m8, _wdown(ds1_w))
    m2d = m2f.reshape(1, D2, D2, D2)
    x2_p = _to_planes(xd.reshape(-1, D2, D2, D2), D2, PL2)
    m2_p = _to_planes(m2d, D2, PL2)
    f0p, out0p = _chain(x2_p, m2_p,
                        [(_w27(db2_w1), None, True, True),
                         (_w27(db2_w2), None, False, True)], D2,
                        head_w=_w27(out0_w))
    f0_d = _from_planes(f0p, D2)

    m1_8 = _sub8(m1d).reshape(8, D2 ** 3)
    y8 = _up2(f0_d.reshape(-1, D2 ** 3), m1_8, _wup(us0_w))
    xup = _interleave8(y8, us0_w.shape[0], D2)
    cat = jnp.concatenate([r1_d, xup], axis=0)
    f1p, out1p = _chain(_to_planes(cat, D1, PL1), m1_p,
                        [(_w27(ub0_w1), None, True, True),
                         (_w27(ub0_w2), None, False, True)], D1,
                        head_w=_w27(out1_w))
    f1_d = _from_planes(f1p, D1)

    m0_8 = _sub8(m0d).reshape(8, D1 ** 3)
    y8 = _up2(f1_d.reshape(-1, D1 ** 3), m0_8, _wup(us1_w))
    xup_p = _to_planes_fat(_interleave8(y8, us1_w.shape[0], D1), D0)
    x_p = _conv_stream_fat([r0_p, xup_p], m0_p, _w27(ub1_w1), D0,
                           inorm=True, lrelu=True, out_dtype=jnp.bfloat16)
    f2p, out2p = _conv_stream_fat([x_p], m0_p, _w27(ub1_w2), D0,
                                  lrelu=True, head_w=_w27(out2_w))

    out0 = _from_planes(out0p, D2)[None]
    out1 = _from_planes(out1p, D1)[None]
    out2 = _from_planes_fat(out2p, D0)[None]
    f0 = f0_d[None]
    f1 = _from_planes(f1p, D1)[None]
    f2 = _from_planes_fat(f2p, D0)[None]
    return ((out0, out1, out2), (f0, f1, f2))


# speedup vs baseline: 1.7604x; 1.0816x over previous
import jax
import jax.numpy as jnp
from jax.experimental import pallas as pl
from jax.experimental.pallas import tpu as pltpu

_F32 = jnp.float32
_PAD = 128
_ROW = 128
_LEAD = 256


def _pl_lanes(D):
    S = D + 2
    need = _PAD + S * S + S + 1
    return -(-need // 128) * 128


def _fat_lanes(D):
    S = D + 2
    return _LEAD + S * _ROW + _LEAD


def _wslab(w):
    CO, CI = w.shape[1], w.shape[2]
    return jnp.transpose(w.reshape(3, 3, 3, CO, CI),
                         (1, 2, 3, 0, 4)).reshape(9, CO, 3 * CI)


def _conv_slab_slim(V, w_ref, C3, S, S2):
    y = None
    for q, (dy, dx) in enumerate([(a, b) for a in (-1, 0, 1) for b in (-1, 0, 1)]):
        o = _PAD + dy * S + dx
        xc = jax.lax.slice(V, (0, o), (C3, o + S2))
        t = jax.lax.dot(w_ref[q], xc, preferred_element_type=_F32)
        y = t if y is None else y + t
    return y


def _conv_slab_fat(V, w_ref, C3, S):
    SP = S * _ROW
    vq = jax.lax.slice(V, (0, _LEAD - 129), (C3, _LEAD - 129 + 256 + SP))
    vp = jax.lax.slice(V, (0, _LEAD - 127), (C3, _LEAD - 127 + 256 + SP))
    y = None
    q = 0
    for dy in (-1, 0, 1):
        s = (dy + 1) * _ROW
        for src, s0 in ((vq, s), (V, _LEAD - 128 + s), (vp, s)):
            xc = jax.lax.slice(src, (0, s0), (C3, s0 + SP))
            t = jax.lax.dot(w_ref[q], xc, preferred_element_type=_F32)
            y = t if y is None else y + t
            q += 1
    return y


def _chain(x, m, layers, D, head_w=None, out_dtype=_F32):
    S = D + 2
    S2 = S * S
    PL = x.shape[2]
    L = len(layers)
    COs = [w.shape[1] for (w, _, _, _) in layers]
    CIs = [x.shape[1]] + COs[:-1]
    wgs = [_wslab(w) for (w, _, _, _) in layers]
    biases = [b.reshape(-1, 1) if b is not None else None
              for (_, b, _, _) in layers]
    use_A = [((L - 1 - i) % 2) == 1 for i in range(L)]
    CA = max([COs[i] for i in range(L) if use_A[i]], default=1)
    if head_w is not None:
        h_wg = _wslab(head_w)

    def body(*refs):
        it = iter(refs)
        x_ref = next(it)
        m_ref = next(it)
        w_refs = [next(it) for _ in range(L)]
        b_refs = [next(it) if b is not None else None for b in biases]
        if head_w is not None:
            hw_ref = next(it)
        o_ref = next(it)
        if head_w is not None:
            h_ref = next(it)
        a_ref = next(it)

        def run_stage(i, src_ref):
            dst = a_ref if use_A[i] else o_ref
            CI_i, CO_i = CIs[i], COs[i]
            _, _, inorm, lrelu = layers[i]
            dst[0] = jnp.zeros((CO_i, PL), dst.dtype)
            dst[S - 1] = jnp.zeros((CO_i, PL), dst.dtype)

            def plane(g, carry):
                s1, s2, nn = carry
                V = src_ref[pl.ds(g, 3)].reshape(3 * CI_i, PL)
                y = _conv_slab_slim(V, w_refs[i], 3 * CI_i, S, S2)
                if b_refs[i] is not None:
                    y = y + b_refs[i][...]
                mm = jax.lax.slice(m_ref[g + 1], (0, _PAD), (1, _PAD + S2))
                y = y * mm
                if lrelu and not inorm:
                    y = jnp.where(y >= 0, y, 0.2 * y)
                dst[g + 1] = jnp.pad(y, ((0, 0), (_PAD, PL - _PAD - S2)))
                if inorm:
                    s1 = s1 + jnp.sum(y, axis=1, keepdims=True)
                    s2 = s2 + jnp.sum(y * y, axis=1, keepdims=True)
                    nn = nn + jnp.sum(mm)
                return (s1, s2, nn)

            init = (jnp.zeros((CO_i, 1), _F32), jnp.zeros((CO_i, 1), _F32),
                    _F32(0))
            s1, s2, nn = jax.lax.fori_loop(0, D, plane, init)
            if inorm:
                n = jnp.maximum(nn, 1.0)
                mu = s1 / n
                var = s2 / n - mu * mu
                inv = jax.lax.rsqrt(var + 1e-5)

                def norm(z, _):
                    v = (dst[z] - mu) * inv * m_ref[z]
                    if lrelu:
                        v = jnp.where(v >= 0, v, 0.2 * v)
                    dst[z] = v
                    return 0

                jax.lax.fori_loop(1, D + 1, norm, 0)
            return dst

        src = x_ref
        for i in range(L):
            src = run_stage(i, src)

        if head_w is not None:
            h_ref[0] = jnp.zeros((1, PL), _F32)
            h_ref[S - 1] = jnp.zeros((1, PL), _F32)

            def hplane(g, _):
                V = o_ref[pl.ds(g, 3)].reshape(3 * COs[-1], PL)
                y = _conv_slab_slim(V, hw_ref, 3 * COs[-1], S, S2)
                mm = jax.lax.slice(m_ref[g + 1], (0, _PAD), (1, _PAD + S2))
                y = y * mm
                h_ref[g + 1] = jnp.pad(y, ((0, 0), (_PAD, PL - _PAD - S2)))
                return 0

            jax.lax.fori_loop(0, D, hplane, 0)

    args = [x, m] + wgs + [b for b in biases if b is not None]
    out_shapes = [jax.ShapeDtypeStruct((S, COs[-1], PL), out_dtype)]
    if head_w is not None:
        args = args + [h_wg]
        out_shapes.append(jax.ShapeDtypeStruct((S, 1, PL), _F32))
    res = pl.pallas_call(
        body,
        out_shape=tuple(out_shapes),
        scratch_shapes=[pltpu.VMEM((S, CA, PL), _F32)],
    )(*args)
    return res if head_w is not None else res[0]


def _conv_stream_fat(xs, m, w, D, bias=None, inorm=False, lrelu=False,
                     head_w=None, out_dtype=_F32):
    S = D + 2
    SP = S * _ROW
    PLf = xs[0].shape[2]
    CIs_ = [x.shape[1] for x in xs]
    CO = w.shape[1]
    off_ci = [sum(CIs_[:j]) for j in range(len(CIs_))]
    wgs = [_wslab(w[:, :, off_ci[j]:off_ci[j] + CIs_[j]]) for j in range(len(CIs_))]
    if head_w is not None:
        hw = _wslab(head_w)

    def body(*refs):
        it = iter(refs)
        x_refs = [[next(it) for _ in range(3)] for _ in CIs_]
        m_ref = next(it)
        w_refs = [next(it) for _ in CIs_]
        b_ref = next(it) if bias is not None else None
        hw_ref = next(it) if head_w is not None else None
        o_ref = next(it)
        h_ref = next(it) if head_w is not None else None
        s1_ref, s2_ref, nn_ref = next(it), next(it), next(it)
        g = pl.program_id(0)

        @pl.when(g == 0)
        def _init():
            s1_ref[...] = jnp.zeros((CO, 1), _F32)
            s2_ref[...] = jnp.zeros((CO, 1), _F32)
            nn_ref[0] = _F32(0)
            o_ref[0] = jnp.zeros((CO, PLf), o_ref.dtype)
            o_ref[S - 1] = jnp.zeros((CO, PLf), o_ref.dtype)
            if head_w is not None:
                h_ref[0] = jnp.zeros((1, PLf), _F32)
                h_ref[S - 1] = jnp.zeros((1, PLf), _F32)

        y = None
        for j in range(len(CIs_)):
            V = jnp.concatenate(
                [x_refs[j][dz][0].astype(_F32) for dz in range(3)], axis=0)
            t = _conv_slab_fat(V, w_refs[j], 3 * CIs_[j], S)
            y = t if y is None else y + t
        if bias is not None:
            y = y + b_ref[...]
        mm = jax.lax.slice(m_ref[g + 1], (0, _LEAD), (1, _LEAD + SP))
        y = y * mm
        if lrelu and not inorm:
            y = jnp.where(y >= 0, y, 0.2 * y)
        o_ref[g + 1] = jnp.pad(y, ((0, 0), (_LEAD, PLf - _LEAD - SP))).astype(o_ref.dtype)
        if inorm:
            s1_ref[...] += jnp.sum(y, axis=1, keepdims=True)
            s2_ref[...] += jnp.sum(y * y, axis=1, keepdims=True)
            nn_ref[0] += jnp.sum(mm)

        @pl.when(g == D - 1)
        def _finish():
            if inorm:
                n = jnp.maximum(nn_ref[0], 1.0)
                mu = s1_ref[...] / n
                var = s2_ref[...] / n - mu * mu
                inv = jax.lax.rsqrt(var + 1e-5)

                def norm(z, _):
                    v = (o_ref[z].astype(_F32) - mu) * inv * m_ref[z]
                    if lrelu:
                        v = jnp.where(v >= 0, v, 0.2 * v)
                    o_ref[z] = v.astype(o_ref.dtype)
                    return 0

                jax.lax.fori_loop(1, D + 1, norm, 0)
            if head_w is not None:
                def hplane(gz, _):
                    V = o_ref[pl.ds(gz, 3)].astype(_F32).reshape(3 * CO, PLf)
                    yh = _conv_slab_fat(V, hw_ref, 3 * CO, S)
                    mmh = jax.lax.slice(m_ref[gz + 1], (0, _LEAD), (1, _LEAD + SP))
                    yh = yh * mmh
                    h_ref[gz + 1] = jnp.pad(
                        yh, ((0, 0), (_LEAD, PLf - _LEAD - SP)))
                    return 0

                jax.lax.fori_loop(0, D, hplane, 0)

    in_specs = []
    args = []
    for j, x in enumerate(xs):
        for d in range(3):
            in_specs.append(pl.BlockSpec((1, CIs_[j], PLf),
                                         (lambda dd: lambda g: (g + dd, 0, 0))(d)))
            args.append(x)
    in_specs.append(pl.BlockSpec((S, 1, PLf), lambda g: (0, 0, 0)))
    args.append(m)
    for wg in wgs:
        in_specs.append(pl.BlockSpec(wg.shape, lambda g: (0, 0, 0)))
        args.append(wg)
    if bias is not None:
        b2 = bias.reshape(CO, 1)
        in_specs.append(pl.BlockSpec(b2.shape, lambda g: (0, 0)))
        args.append(b2)
    if head_w is not None:
        in_specs.append(pl.BlockSpec(hw.shape, lambda g: (0, 0, 0)))
        args.append(hw)
    out_shapes = [jax.ShapeDtypeStruct((S, CO, PLf), out_dtype)]
    out_specs = [pl.BlockSpec((S, CO, PLf), lambda g: (0, 0, 0))]
    if head_w is not None:
        out_shapes.append(jax.ShapeDtypeStruct((S, 1, PLf), _F32))
        out_specs.append(pl.BlockSpec((S, 1, PLf), lambda g: (0, 0, 0)))
    res = pl.pallas_call(
        body,
        grid=(D,),
        in_specs=in_specs,
        out_specs=out_specs,
        scratch_shapes=[
            pltpu.VMEM((CO, 1), _F32),
            pltpu.VMEM((CO, 1), _F32),
            pltpu.SMEM((1,), _F32),
        ],
        out_shape=tuple(out_shapes),
    )(*args)
    return res if head_w is not None else res[0]


def _down2(x8, m8, w):
    CO = w.shape[0]
    Nc = x8.shape[1]

    def body(x_ref, m_ref, w_ref, o_ref, mo_ref):
        mo = jnp.max(m_ref[...], axis=0, keepdims=True)
        y = jax.lax.dot(w_ref[...], x_ref[...].astype(_F32),
                        preferred_element_type=_F32)
        o_ref[...] = y * mo
        mo_ref[...] = mo

    return pl.pallas_call(
        body,
        out_shape=(jax.ShapeDtypeStruct((CO, Nc), _F32),
                   jax.ShapeDtypeStruct((1, Nc), _F32)))(x8, m8, w)


def _up2(f, m8, w8):
    CO = w8.shape[1]
    Nc = f.shape[1]

    def body(f_ref, m_ref, w_ref, o_ref):
        for a in range(8):
            y = jax.lax.dot(w_ref[a], f_ref[...], preferred_element_type=_F32)
            o_ref[a] = (y * m_ref[a:a + 1, :]).astype(o_ref.dtype)

    return pl.pallas_call(
        body, out_shape=jax.ShapeDtypeStruct((8, CO, Nc), jnp.bfloat16))(f, m8, w8)


def _to_planes(x, D, PL):
    C = x.shape[0]
    S = D + 2
    xp = jnp.pad(x, ((0, 0), (1, 1), (1, 1), (1, 1)))
    xp = jnp.transpose(xp.reshape(C, S, S * S), (1, 0, 2))
    return jnp.pad(xp, ((0, 0), (0, 0), (_PAD, PL - _PAD - S * S)))


def _from_planes(x, D):
    S = D + 2
    v = x[1:D + 1, :, _PAD:_PAD + S * S]
    v = jnp.transpose(v, (1, 0, 2)).reshape(-1, D, S, S)
    return v[:, :, 1:D + 1, 1:D + 1]


def _to_planes_fat(x, D):
    C = x.shape[0]
    S = D + 2
    xp = jnp.pad(x, ((0, 0), (1, 1), (1, 1), (1, _ROW - D - 1)))
    xp = jnp.transpose(xp.reshape(C, S, S * _ROW), (1, 0, 2))
    return jnp.pad(xp, ((0, 0), (0, 0), (_LEAD, _LEAD)))


def _from_planes_fat(x, D):
    S = D + 2
    v = x[1:D + 1, :, _LEAD:_LEAD + S * _ROW]
    v = jnp.transpose(v, (1, 0, 2)).reshape(-1, D, S, _ROW)
    return v[:, :, 1:D + 1, 1:D + 1]


def _sub8(x):
    C, D = x.shape[0], x.shape[1]
    h = D // 2
    y = x.reshape(C, h, 2, h, 2, h, 2)
    y = jnp.transpose(y, (2, 4, 6, 0, 1, 3, 5))
    return y.reshape(8, C, h * h * h)


def _interleave8(y8, CO, h):
    y = y8.reshape(2, 2, 2, CO, h, h, h)
    y = jnp.transpose(y, (3, 4, 0, 5, 1, 6, 2))
    return y.reshape(CO, 2 * h, 2 * h, 2 * h)


def _w27(w):
    return jnp.transpose(w, (2, 3, 4, 0, 1)).reshape(27, w.shape[0], w.shape[1])


def _wdown(w):
    return jnp.transpose(w, (0, 2, 3, 4, 1)).reshape(w.shape[0], 8 * w.shape[1])


def _wup(w):
    return jnp.transpose(w[:, :, ::-1, ::-1, ::-1],
                         (2, 3, 4, 0, 1)).reshape(8, w.shape[0], w.shape[1])


def kernel(voxel_feats, voxel_mask, W_init, b_init, db0_w1, db0_w2, db1_w1,
           db1_w2, db2_w1, db2_w2, ds0_w, ds1_w, us0_w, us1_w, ub0_w1, ub0_w2,
           ub1_w1, ub1_w2, out0_w, out1_w, out2_w):
    vf = voxel_feats[0].astype(_F32)
    m0d = voxel_mask[0].astype(_F32)
    D0 = vf.shape[-1]
    D1, D2 = D0 // 2, D0 // 4
    PL1, PL2 = _pl_lanes(D1), _pl_lanes(D2)

    vf_p = _to_planes_fat(vf, D0)
    m0_p = _to_planes_fat(m0d, D0)
    x = _conv_stream_fat([vf_p], m0_p, _w27(W_init), D0, bias=b_init)
    x = _conv_stream_fat([x], m0_p, _w27(db0_w1), D0, inorm=True, lrelu=True)
    r0_p = _conv_stream_fat([x], m0_p, _w27(db0_w2), D0, lrelu=True)
    r0_d = _from_planes_fat(r0_p, D0)

    x8 = _sub8(r0_d).reshape(8 * r0_d.shape[0], D1 ** 3)
    m8 = _sub8(m0d).reshape(8, D1 ** 3)
    xd, m1f = _down2(x8, m8, _wdown(ds0_w))
    m1d = m1f.reshape(1, D1, D1, D1)
    x1_p = _to_planes(xd.reshape(-1, D1, D1, D1), D1, PL1)
    m1_p = _to_planes(m1d, D1, PL1)
    r1_p = _chain(x1_p, m1_p,
                  [(_w27(db1_w1), None, True, True),
                   (_w27(db1_w2), None, False, True)], D1)
    r1_d = _from_planes(r1_p, D1)

    x8 = _sub8(r1_d).reshape(8 * r1_d.shape[0], D2 ** 3)
    m8 = _sub8(m1d).reshape(8, D2 ** 3)
    xd, m2f = _down2(x8, m8, _wdown(ds1_w))
    m2d = m2f.reshape(1, D2, D2, D2)
    x2_p = _to_planes(xd.reshape(-1, D2, D2, D2), D2, PL2)
    m2_p = _to_planes(m2d, D2, PL2)
    f0p, out0p = _chain(x2_p, m2_p,
                        [(_w27(db2_w1), None, True, True),
                         (_w27(db2_w2), None, False, True)], D2,
                        head_w=_w27(out0_w))
    f0_d = _from_planes(f0p, D2)

    m1_8 = _sub8(m1d).reshape(8, D2 ** 3)
    y8 = _up2(f0_d.reshape(-1, D2 ** 3), m1_8, _wup(us0_w))
    xup = _interleave8(y8, us0_w.shape[0], D2)
    cat = jnp.concatenate([r1_d, xup], axis=0)
    f1p, out1p = _chain(_to_planes(cat, D1, PL1), m1_p,
                        [(_w27(ub0_w1), None, True, True),
                         (_w27(ub0_w2), None, False, True)], D1,
                        head_w=_w27(out1_w))
    f1_d = _from_planes(f1p, D1)

    m0_8 = _sub8(m0d).reshape(8, D1 ** 3)
    y8 = _up2(f1_d.reshape(-1, D1 ** 3), m0_8, _wup(us1_w))
    xup_p = _to_planes_fat(_interleave8(y8, us1_w.shape[0], D1), D0)
    x_p = _conv_stream_fat([r0_p, xup_p], m0_p, _w27(ub1_w1), D0,
                           inorm=True, lrelu=True, out_dtype=jnp.bfloat16)
    f2p, out2p = _conv_stream_fat([x_p], m0_p, _w27(ub1_w2), D0,
                                  lrelu=True, head_w=_w27(out2_w))

    out0 = _from_planes(out0p, D2)[None]
    out1 = _from_planes(out1p, D1)[None]
    out2 = _from_planes_fat(out2p, D0)[None]
    f0 = f0_d[None]
    f1 = _from_planes(f1p, D1)[None]
    f2 = _from_planes_fat(f2p, D0)[None]
    return ((out0, out1, out2), (f0, f1, f2))


# speedup vs baseline: 1.8885x; 1.0727x over previous
import jax
import jax.numpy as jnp
from jax.experimental import pallas as pl
from jax.experimental.pallas import tpu as pltpu

_F32 = jnp.float32
_PAD = 128
_ROW = 128
_LEAD = 256


def _pl_lanes(D):
    S = D + 2
    need = _PAD + S * S + S + 1
    return -(-need // 128) * 128


def _fat_lanes(D):
    S = D + 2
    return _LEAD + S * _ROW + _LEAD


def _wslab(w):
    CO, CI = w.shape[1], w.shape[2]
    return jnp.transpose(w.reshape(3, 3, 3, CO, CI),
                         (1, 2, 3, 0, 4)).reshape(9, CO, 3 * CI)


def _conv_slab_slim(V, w_ref, C3, S, S2):
    y = None
    for q, (dy, dx) in enumerate([(a, b) for a in (-1, 0, 1) for b in (-1, 0, 1)]):
        o = _PAD + dy * S + dx
        xc = jax.lax.slice(V, (0, o), (C3, o + S2))
        t = jax.lax.dot(w_ref[q], xc, preferred_element_type=_F32)
        y = t if y is None else y + t
    return y


def _conv_slab_fat(V, w_ref, C3, S):
    SP = S * _ROW
    vq = jax.lax.slice(V, (0, _LEAD - 129), (C3, _LEAD - 129 + 256 + SP))
    vp = jax.lax.slice(V, (0, _LEAD - 127), (C3, _LEAD - 127 + 256 + SP))
    y = None
    q = 0
    for dy in (-1, 0, 1):
        s = (dy + 1) * _ROW
        for src, s0 in ((vq, s), (V, _LEAD - 128 + s), (vp, s)):
            xc = jax.lax.slice(src, (0, s0), (C3, s0 + SP))
            t = jax.lax.dot(w_ref[q], xc, preferred_element_type=_F32)
            y = t if y is None else y + t
            q += 1
    return y


def _chain(x, m, layers, D, head_w=None, out_dtype=_F32):
    S = D + 2
    S2 = S * S
    PL = x.shape[2]
    L = len(layers)
    COs = [w.shape[1] for (w, _, _, _) in layers]
    CIs = [x.shape[1]] + COs[:-1]
    wgs = [_wslab(w) for (w, _, _, _) in layers]
    biases = [b.reshape(-1, 1) if b is not None else None
              for (_, b, _, _) in layers]
    use_A = [((L - 1 - i) % 2) == 1 for i in range(L)]
    CA = max([COs[i] for i in range(L) if use_A[i]], default=1)
    if head_w is not None:
        h_wg = _wslab(head_w)

    def body(*refs):
        it = iter(refs)
        x_ref = next(it)
        m_ref = next(it)
        w_refs = [next(it) for _ in range(L)]
        b_refs = [next(it) if b is not None else None for b in biases]
        if head_w is not None:
            hw_ref = next(it)
        o_ref = next(it)
        if head_w is not None:
            h_ref = next(it)
        a_ref = next(it)

        def run_stage(i, src_ref):
            dst = a_ref if use_A[i] else o_ref
            CI_i, CO_i = CIs[i], COs[i]
            _, _, inorm, lrelu = layers[i]
            dst[0] = jnp.zeros((CO_i, PL), dst.dtype)
            dst[S - 1] = jnp.zeros((CO_i, PL), dst.dtype)

            def plane(g, carry):
                s1, s2, nn = carry
                V = src_ref[pl.ds(g, 3)].reshape(3 * CI_i, PL)
                y = _conv_slab_slim(V, w_refs[i], 3 * CI_i, S, S2)
                if b_refs[i] is not None:
                    y = y + b_refs[i][...]
                mm = jax.lax.slice(m_ref[g + 1], (0, _PAD), (1, _PAD + S2))
                y = y * mm
                if lrelu and not inorm:
                    y = jnp.where(y >= 0, y, 0.2 * y)
                dst[g + 1] = jnp.pad(y, ((0, 0), (_PAD, PL - _PAD - S2)))
                if inorm:
                    s1 = s1 + jnp.sum(y, axis=1, keepdims=True)
                    s2 = s2 + jnp.sum(y * y, axis=1, keepdims=True)
                    nn = nn + jnp.sum(mm)
                return (s1, s2, nn)

            init = (jnp.zeros((CO_i, 1), _F32), jnp.zeros((CO_i, 1), _F32),
                    _F32(0))
            s1, s2, nn = jax.lax.fori_loop(0, D, plane, init)
            if inorm:
                n = jnp.maximum(nn, 1.0)
                mu = s1 / n
                var = s2 / n - mu * mu
                inv = jax.lax.rsqrt(var + 1e-5)

                def norm(z, _):
                    v = (dst[z] - mu) * inv * m_ref[z]
                    if lrelu:
                        v = jnp.where(v >= 0, v, 0.2 * v)
                    dst[z] = v
                    return 0

                jax.lax.fori_loop(1, D + 1, norm, 0)
            return dst

        src = x_ref
        for i in range(L):
            src = run_stage(i, src)

        if head_w is not None:
            h_ref[0] = jnp.zeros((1, PL), _F32)
            h_ref[S - 1] = jnp.zeros((1, PL), _F32)

            def hplane(g, _):
                V = o_ref[pl.ds(g, 3)].reshape(3 * COs[-1], PL)
                y = _conv_slab_slim(V, hw_ref, 3 * COs[-1], S, S2)
                mm = jax.lax.slice(m_ref[g + 1], (0, _PAD), (1, _PAD + S2))
                y = y * mm
                h_ref[g + 1] = jnp.pad(y, ((0, 0), (_PAD, PL - _PAD - S2)))
                return 0

            jax.lax.fori_loop(0, D, hplane, 0)

    args = [x, m] + wgs + [b for b in biases if b is not None]
    out_shapes = [jax.ShapeDtypeStruct((S, COs[-1], PL), out_dtype)]
    if head_w is not None:
        args = args + [h_wg]
        out_shapes.append(jax.ShapeDtypeStruct((S, 1, PL), _F32))
    res = pl.pallas_call(
        body,
        out_shape=tuple(out_shapes),
        scratch_shapes=[pltpu.VMEM((S, CA, PL), _F32)],
    )(*args)
    return res if head_w is not None else res[0]


def _conv_stream_fat(xs, m, w, D, bias=None, inorm=False, lrelu=False,
                     head_w=None, out_dtype=_F32):
    S = D + 2
    SP = S * _ROW
    PLf = xs[0].shape[2]
    CIs_ = [x.shape[1] for x in xs]
    CO = w.shape[1]
    off_ci = [sum(CIs_[:j]) for j in range(len(CIs_))]
    wgs = [_wslab(w[:, :, off_ci[j]:off_ci[j] + CIs_[j]]) for j in range(len(CIs_))]
    if head_w is not None:
        hw = _wslab(head_w)

    ZB = 8 if len(xs) == 1 else 4
    NG = D // ZB

    def body(*refs):
        it = iter(refs)
        x_refs = [[next(it) for _ in range(3)] for _ in CIs_]
        m_ref = next(it)
        w_refs = [next(it) for _ in CIs_]
        b_ref = next(it) if bias is not None else None
        hw_ref = next(it) if head_w is not None else None
        o_ref = next(it)
        h_ref = next(it) if head_w is not None else None
        s1_ref, s2_ref, nn_ref = next(it), next(it), next(it)
        g = pl.program_id(0)

        @pl.when(g == 0)
        def _init():
            s1_ref[...] = jnp.zeros((CO, 1), _F32)
            s2_ref[...] = jnp.zeros((CO, 1), _F32)
            nn_ref[0] = _F32(0)
            o_ref[0] = jnp.zeros((CO, PLf), o_ref.dtype)
            o_ref[S - 1] = jnp.zeros((CO, PLf), o_ref.dtype)
            if head_w is not None:
                h_ref[0] = jnp.zeros((1, PLf), _F32)
                h_ref[S - 1] = jnp.zeros((1, PLf), _F32)

        ts1 = jnp.zeros((CO, 1), _F32)
        ts2 = jnp.zeros((CO, 1), _F32)
        tnn = _F32(0)
        for j in range(len(CIs_)):
            x_refs[j] = [x_refs[j][0][...], x_refs[j][1][0], x_refs[j][2][0]]
        for k in range(ZB):
            y = None
            for j in range(len(CIs_)):
                slab, h0, h1 = x_refs[j]
                CI_j = CIs_[j]
                if k <= ZB - 3:
                    V = slab[k:k + 3].reshape(3 * CI_j, PLf)
                elif k == ZB - 2:
                    V = jnp.concatenate(
                        [slab[k:k + 2].reshape(2 * CI_j, PLf), h0], axis=0)
                else:
                    V = jnp.concatenate([slab[k], h0, h1], axis=0)
                t = _conv_slab_fat(V.astype(_F32), w_refs[j], 3 * CI_j, S)
                y = t if y is None else y + t
            if bias is not None:
                y = y + b_ref[...]
            z = g * ZB + k + 1
            mm = jax.lax.slice(m_ref[z], (0, _LEAD), (1, _LEAD + SP))
            y = y * mm
            if lrelu and not inorm:
                y = jnp.where(y >= 0, y, 0.2 * y)
            o_ref[z] = jnp.pad(y, ((0, 0), (_LEAD, PLf - _LEAD - SP))).astype(o_ref.dtype)
            if inorm:
                ts1 = ts1 + jnp.sum(y, axis=1, keepdims=True)
                ts2 = ts2 + jnp.sum(y * y, axis=1, keepdims=True)
                tnn = tnn + jnp.sum(mm)
        if inorm:
            s1_ref[...] += ts1
            s2_ref[...] += ts2
            nn_ref[0] += tnn

        @pl.when(g == NG - 1)
        def _finish():
            if inorm:
                n = jnp.maximum(nn_ref[0], 1.0)
                mu = s1_ref[...] / n
                var = s2_ref[...] / n - mu * mu
                inv = jax.lax.rsqrt(var + 1e-5)

                def norm(z, _):
                    v = (o_ref[z].astype(_F32) - mu) * inv * m_ref[z]
                    if lrelu:
                        v = jnp.where(v >= 0, v, 0.2 * v)
                    o_ref[z] = v.astype(o_ref.dtype)
                    return 0

                jax.lax.fori_loop(1, D + 1, norm, 0)
            if head_w is not None:
                def hplane(gz, _):
                    V = o_ref[pl.ds(gz, 3)].astype(_F32).reshape(3 * CO, PLf)
                    yh = _conv_slab_fat(V, hw_ref, 3 * CO, S)
                    mmh = jax.lax.slice(m_ref[gz + 1], (0, _LEAD), (1, _LEAD + SP))
                    yh = yh * mmh
                    h_ref[gz + 1] = jnp.pad(
                        yh, ((0, 0), (_LEAD, PLf - _LEAD - SP)))
                    return 0

                jax.lax.fori_loop(0, D, hplane, 0)

    in_specs = []
    args = []
    for j, x in enumerate(xs):
        in_specs.append(pl.BlockSpec((ZB, CIs_[j], PLf), lambda g: (g, 0, 0)))
        args.append(x)
        for d in range(2):
            in_specs.append(pl.BlockSpec(
                (1, CIs_[j], PLf),
                (lambda dd: lambda g: (g * ZB + ZB + dd, 0, 0))(d)))
            args.append(x)
    in_specs.append(pl.BlockSpec((S, 1, PLf), lambda g: (0, 0, 0)))
    args.append(m)
    for wg in wgs:
        in_specs.append(pl.BlockSpec(wg.shape, lambda g: (0, 0, 0)))
        args.append(wg)
    if bias is not None:
        b2 = bias.reshape(CO, 1)
        in_specs.append(pl.BlockSpec(b2.shape, lambda g: (0, 0)))
        args.append(b2)
    if head_w is not None:
        in_specs.append(pl.BlockSpec(hw.shape, lambda g: (0, 0, 0)))
        args.append(hw)
    out_shapes = [jax.ShapeDtypeStruct((S, CO, PLf), out_dtype)]
    out_specs = [pl.BlockSpec((S, CO, PLf), lambda g: (0, 0, 0))]
    if head_w is not None:
        out_shapes.append(jax.ShapeDtypeStruct((S, 1, PLf), _F32))
        out_specs.append(pl.BlockSpec((S, 1, PLf), lambda g: (0, 0, 0)))
    res = pl.pallas_call(
        body,
        grid=(NG,),
        in_specs=in_specs,
        out_specs=out_specs,
        scratch_shapes=[
            pltpu.VMEM((CO, 1), _F32),
            pltpu.VMEM((CO, 1), _F32),
            pltpu.SMEM((1,), _F32),
        ],
        out_shape=tuple(out_shapes),
    )(*args)
    return res if head_w is not None else res[0]


def _down2(x8, m8, w):
    CO = w.shape[0]
    Nc = x8.shape[1]

    def body(x_ref, m_ref, w_ref, o_ref, mo_ref):
        mo = jnp.max(m_ref[...], axis=0, keepdims=True)
        y = jax.lax.dot(w_ref[...], x_ref[...].astype(_F32),
                        preferred_element_type=_F32)
        o_ref[...] = y * mo
        mo_ref[...] = mo

    return pl.pallas_call(
        body,
        out_shape=(jax.ShapeDtypeStruct((CO, Nc), _F32),
                   jax.ShapeDtypeStruct((1, Nc), _F32)))(x8, m8, w)


def _up2(f, m8, w8):
    CO = w8.shape[1]
    Nc = f.shape[1]

    def body(f_ref, m_ref, w_ref, o_ref):
        for a in range(8):
            y = jax.lax.dot(w_ref[a], f_ref[...], preferred_element_type=_F32)
            o_ref[a] = (y * m_ref[a:a + 1, :]).astype(o_ref.dtype)

    return pl.pallas_call(
        body, out_shape=jax.ShapeDtypeStruct((8, CO, Nc), jnp.bfloat16))(f, m8, w8)


def _to_planes(x, D, PL):
    C = x.shape[0]
    S = D + 2
    xp = jnp.pad(x, ((0, 0), (1, 1), (1, 1), (1, 1)))
    xp = jnp.transpose(xp.reshape(C, S, S * S), (1, 0, 2))
    return jnp.pad(xp, ((0, 0), (0, 0), (_PAD, PL - _PAD - S * S)))


def _from_planes(x, D):
    S = D + 2
    v = x[1:D + 1, :, _PAD:_PAD + S * S]
    v = jnp.transpose(v, (1, 0, 2)).reshape(-1, D, S, S)
    return v[:, :, 1:D + 1, 1:D + 1]


def _to_planes_fat(x, D):
    C = x.shape[0]
    S = D + 2
    xp = jnp.pad(x, ((0, 0), (1, 1), (1, 1), (1, _ROW - D - 1)))
    xp = jnp.transpose(xp.reshape(C, S, S * _ROW), (1, 0, 2))
    return jnp.pad(xp, ((0, 0), (0, 0), (_LEAD, _LEAD)))


def _from_planes_fat(x, D):
    S = D + 2
    v = x[1:D + 1, :, _LEAD:_LEAD + S * _ROW]
    v = jnp.transpose(v, (1, 0, 2)).reshape(-1, D, S, _ROW)
    return v[:, :, 1:D + 1, 1:D + 1]


def _sub8(x):
    C, D = x.shape[0], x.shape[1]
    h = D // 2
    y = x.reshape(C, h, 2, h, 2, h, 2)
    y = jnp.transpose(y, (2, 4, 6, 0, 1, 3, 5))
    return y.reshape(8, C, h * h * h)


def _interleave8(y8, CO, h):
    y = y8.reshape(2, 2, 2, CO, h, h, h)
    y = jnp.transpose(y, (3, 4, 0, 5, 1, 6, 2))
    return y.reshape(CO, 2 * h, 2 * h, 2 * h)


def _w27(w):
    return jnp.transpose(w, (2, 3, 4, 0, 1)).reshape(27, w.shape[0], w.shape[1])


def _wdown(w):
    return jnp.transpose(w, (0, 2, 3, 4, 1)).reshape(w.shape[0], 8 * w.shape[1])


def _wup(w):
    return jnp.transpose(w[:, :, ::-1, ::-1, ::-1],
                         (2, 3, 4, 0, 1)).reshape(8, w.shape[0], w.shape[1])


def kernel(voxel_feats, voxel_mask, W_init, b_init, db0_w1, db0_w2, db1_w1,
           db1_w2, db2_w1, db2_w2, ds0_w, ds1_w, us0_w, us1_w, ub0_w1, ub0_w2,
           ub1_w1, ub1_w2, out0_w, out1_w, out2_w):
    vf = voxel_feats[0].astype(_F32)
    m0d = voxel_mask[0].astype(_F32)
    D0 = vf.shape[-1]
    D1, D2 = D0 // 2, D0 // 4
    PL1, PL2 = _pl_lanes(D1), _pl_lanes(D2)

    vf_p = _to_planes_fat(vf, D0)
    m0_p = _to_planes_fat(m0d, D0)
    x = _conv_stream_fat([vf_p], m0_p, _w27(W_init), D0, bias=b_init)
    x = _conv_stream_fat([x], m0_p, _w27(db0_w1), D0, inorm=True, lrelu=True)
    r0_p = _conv_stream_fat([x], m0_p, _w27(db0_w2), D0, lrelu=True)
    r0_d = _from_planes_fat(r0_p, D0)

    x8 = _sub8(r0_d).reshape(8 * r0_d.shape[0], D1 ** 3)
    m8 = _sub8(m0d).reshape(8, D1 ** 3)
    xd, m1f = _down2(x8, m8, _wdown(ds0_w))
    m1d = m1f.reshape(1, D1, D1, D1)
    x1_p = _to_planes(xd.reshape(-1, D1, D1, D1), D1, PL1)
    m1_p = _to_planes(m1d, D1, PL1)
    r1_p = _chain(x1_p, m1_p,
                  [(_w27(db1_w1), None, True, True),
                   (_w27(db1_w2), None, False, True)], D1)
    r1_d = _from_planes(r1_p, D1)

    x8 = _sub8(r1_d).reshape(8 * r1_d.shape[0], D2 ** 3)
    m8 = _sub8(m1d).reshape(8, D2 ** 3)
    xd, m2f = _down2(x8, m8, _wdown(ds1_w))
    m2d = m2f.reshape(1, D2, D2, D2)
    x2_p = _to_planes(xd.reshape(-1, D2, D2, D2), D2, PL2)
    m2_p = _to_planes(m2d, D2, PL2)
    f0p, out0p = _chain(x2_p, m2_p,
                        [(_w27(db2_w1), None, True, True),
                         (_w27(db2_w2), None, False, True)], D2,
                        head_w=_w27(out0_w))
    f0_d = _from_planes(f0p, D2)

    m1_8 = _sub8(m1d).reshape(8, D2 ** 3)
    y8 = _up2(f0_d.reshape(-1, D2 ** 3), m1_8, _wup(us0_w))
    xup = _interleave8(y8, us0_w.shape[0], D2)
    cat = jnp.concatenate([r1_d, xup], axis=0)
    f1p, out1p = _chain(_to_planes(cat, D1, PL1), m1_p,
                        [(_w27(ub0_w1), None, True, True),
                         (_w27(ub0_w2), None, False, True)], D1,
                        head_w=_w27(out1_w))
    f1_d = _from_planes(f1p, D1)

    m0_8 = _sub8(m0d).reshape(8, D1 ** 3)
    y8 = _up2(f1_d.reshape(-1, D1 ** 3), m0_8, _wup(us1_w))
    xup_p = _to_planes_fat(_interleave8(y8, us1_w.shape[0], D1), D0)
    x_p = _conv_stream_fat([r0_p, xup_p], m0_p, _w27(ub1_w1), D0,
                           inorm=True, lrelu=True, out_dtype=jnp.bfloat16)
    f2p, out2p = _conv_stream_fat([x_p], m0_p, _w27(ub1_w2), D0,
                                  lrelu=True, head_w=_w27(out2_w))

    out0 = _from_planes(out0p, D2)[None]
    out1 = _from_planes(out1p, D1)[None]
    out2 = _from_planes_fat(out2p, D0)[None]
    f0 = f0_d[None]
    f1 = _from_planes(f1p, D1)[None]
    f2 = _from_planes_fat(f2p, D0)[None]
    return ((out0, out1, out2), (f0, f1, f2))
